# Initial kernel scaffold; baseline (speedup 1.0000x reference)
import jax
import jax.numpy as jnp
from jax.experimental import pallas as pl


def kernel(x, edge_index, edge_attr, u, W_in, b_in, We1, be1, We2, be2, W_root, b_conv, W_ih, W_hh, b_ih, b_hh, Wd1, bd1, Wd2, bd2, Wd3, bd3):
    raise NotImplementedError("write your pallas kernel here")


# speedup vs baseline: 1.9351x; 1.9351x over previous
import jax
import jax.numpy as jnp
from jax import lax
from jax.experimental import pallas as pl
from jax.experimental.pallas import tpu as pltpu
from jax.experimental.pallas import tpu_sc as plsc

N = 10000
E = 160000
H = 32
HH = H * H
W128 = 128
NODE_DIM = 12
GLOBAL_DIM = 11
EDGE_DIM = 5

NC, NS = 2, 16
NW = NC * NS
CH = 128
NCHUNK = E // CH
CPW = NCHUNK // NW
CEXTRA = NCHUNK - CPW * NW
STRIPE = 632

EB = 1000
EB2 = 2000
NB = 1000

_MESH = plsc.VectorSubcoreMesh(
    core_axis_name="c", subcore_axis_name="s", num_cores=NC, num_subcores=NS)


def _dot(a, b):
    return jnp.dot(a, b, preferred_element_type=jnp.float32)


def _pad128(v, rows):
    return jnp.concatenate(
        [v, jnp.zeros((rows, W128 - H), jnp.float32)], axis=1)


def _gather_body(table_hbm, idx_hbm, out_hbm, idx_v, rows_v, sem):
    c = lax.axis_index("c")
    s = lax.axis_index("s")
    w = c * NS + s

    @pl.loop(0, CPW)
    def _chunk(j):
        off = (j * NW + w) * CH
        pltpu.sync_copy(idx_hbm.at[pl.ds(off, CH)], idx_v)
        pltpu.async_copy(table_hbm.at[idx_v], rows_v, sem).wait()
        pltpu.sync_copy(rows_v, out_hbm.at[pl.ds(off, CH)])

    @pl.when(w < CEXTRA)
    def _extra():
        off = (CPW * NW + w) * CH
        pltpu.sync_copy(idx_hbm.at[pl.ds(off, CH)], idx_v)
        pltpu.async_copy(table_hbm.at[idx_v], rows_v, sem).wait()
        pltpu.sync_copy(rows_v, out_hbm.at[pl.ds(off, CH)])


def _sc_gather(table, idx):
    return pl.kernel(
        _gather_body,
        out_type=jax.ShapeDtypeStruct((E, W128), jnp.float32),
        mesh=_MESH,
        scratch_types=[
            pltpu.VMEM((CH,), jnp.int32),
            pltpu.VMEM((CH, W128), jnp.float32),
            pltpu.SemaphoreType.DMA,
        ],
    )(table, idx)


def _scatter_body(msg_hbm, idx_hbm, zeros_hbm, out_hbm, idx_v, rows_v, acc_sh):
    c = lax.axis_index("c")
    s = lax.axis_index("s")
    w = c * NS + s
    stripe_off = jnp.minimum(s * STRIPE, N - STRIPE)
    pltpu.sync_copy(zeros_hbm.at[pl.ds(stripe_off, STRIPE)],
                    acc_sh.at[pl.ds(stripe_off, STRIPE)])
    plsc.subcore_barrier()

    @pl.loop(0, CPW)
    def _chunk(j):
        off = (j * NW + w) * CH
        pltpu.sync_copy(idx_hbm.at[pl.ds(off, CH)], idx_v)
        pltpu.sync_copy(msg_hbm.at[pl.ds(off, CH)], rows_v)
        pltpu.sync_copy(rows_v, acc_sh.at[idx_v], add=True)

    @pl.when(w < CEXTRA)
    def _extra():
        off = (CPW * NW + w) * CH
        pltpu.sync_copy(idx_hbm.at[pl.ds(off, CH)], idx_v)
        pltpu.sync_copy(msg_hbm.at[pl.ds(off, CH)], rows_v)
        pltpu.sync_copy(rows_v, acc_sh.at[idx_v], add=True)

    plsc.subcore_barrier()
    pltpu.sync_copy(acc_sh.at[pl.ds(stripe_off, STRIPE)],
                    out_hbm.at[c].at[pl.ds(stripe_off, STRIPE)])


def _sc_scatter(msg, idx, zeros_nw):
    return pl.kernel(
        _scatter_body,
        out_type=jax.ShapeDtypeStruct((NC, N, W128), jnp.float32),
        mesh=_MESH,
        scratch_types=[
            pltpu.VMEM((CH,), jnp.int32),
            pltpu.VMEM((CH, W128), jnp.float32),
            pltpu.VMEM_SHARED((N, W128), jnp.float32),
        ],
    )(msg, idx, zeros_nw)


def _counts_body(idx_hbm, ones_hbm, zeros_hbm, out_hbm, idx_v, ones_v, acc_sh):
    c = lax.axis_index("c")
    s = lax.axis_index("s")
    w = c * NS + s
    stripe_off = jnp.minimum(s * STRIPE, N - STRIPE)
    pltpu.sync_copy(zeros_hbm.at[pl.ds(stripe_off, STRIPE)],
                    acc_sh.at[pl.ds(stripe_off, STRIPE)])
    pltpu.sync_copy(ones_hbm, ones_v)
    plsc.subcore_barrier()

    @pl.loop(0, CPW)
    def _chunk(j):
        off = (j * NW + w) * CH
        pltpu.sync_copy(idx_hbm.at[pl.ds(off, CH)], idx_v)
        pltpu.sync_copy(ones_v, acc_sh.at[idx_v], add=True)

    @pl.when(w < CEXTRA)
    def _extra():
        off = (CPW * NW + w) * CH
        pltpu.sync_copy(idx_hbm.at[pl.ds(off, CH)], idx_v)
        pltpu.sync_copy(ones_v, acc_sh.at[idx_v], add=True)

    plsc.subcore_barrier()
    pltpu.sync_copy(acc_sh.at[pl.ds(stripe_off, STRIPE)],
                    out_hbm.at[c].at[pl.ds(stripe_off, STRIPE)])


def _sc_counts(idx, ones_cw, zeros_nw):
    return pl.kernel(
        _counts_body,
        out_type=jax.ShapeDtypeStruct((NC, N, W128), jnp.float32),
        mesh=_MESH,
        scratch_types=[
            pltpu.VMEM((CH,), jnp.int32),
            pltpu.VMEM((CH, W128), jnp.float32),
            pltpu.VMEM_SHARED((N, W128), jnp.float32),
        ],
    )(idx, ones_cw, zeros_nw)


def _prep_body(x_ref, u_ref, wx_ref, wu_ref, b_ref, cnt_ref, h0_ref, inv_ref):
    ub = _dot(u_ref[...], wu_ref[...]) + b_ref[...]
    h0 = jnp.tanh(_dot(x_ref[...], wx_ref[...]) + ub)
    h0_ref[...] = _pad128(h0, NB)
    cnt = jnp.maximum(cnt_ref[0, :, 0:1] + cnt_ref[1, :, 0:1], 1.0)
    inv_ref[...] = jnp.broadcast_to(1.0 / cnt, (NB, H))


def _tc_prep(x, u, wx, wu, b_in2, cnt_parts):
    return pl.pallas_call(
        _prep_body,
        grid=(N // NB,),
        in_specs=[
            pl.BlockSpec((NB, NODE_DIM), lambda i: (i, 0)),
            pl.BlockSpec((1, GLOBAL_DIM), lambda i: (0, 0)),
            pl.BlockSpec((NODE_DIM, H), lambda i: (0, 0)),
            pl.BlockSpec((GLOBAL_DIM, H), lambda i: (0, 0)),
            pl.BlockSpec((1, H), lambda i: (0, 0)),
            pl.BlockSpec((NC, NB, W128), lambda i: (0, i, 0)),
        ],
        out_specs=[
            pl.BlockSpec((NB, W128), lambda i: (i, 0)),
            pl.BlockSpec((NB, H), lambda i: (i, 0)),
        ],
        out_shape=[
            jax.ShapeDtypeStruct((N, W128), jnp.float32),
            jax.ShapeDtypeStruct((N, H), jnp.float32),
        ],
    )(x, u, wx, wu, b_in2, cnt_parts)


def _msg_body(ea_ref, hs_ref, we1_ref, be1_ref, rsel_ref, ssel_ref,
              w2p_ref, be2m_ref, msg_ref):
    z = jnp.maximum(_dot(ea_ref[...], we1_ref[...]) + be1_ref[...], 0.0)
    hs = hs_ref[:, :H]
    a = _dot(z, rsel_ref[...])
    b = _dot(hs, ssel_ref[...])
    p = a * b
    msg = _dot(p, w2p_ref[...]) + _dot(hs, be2m_ref[...])
    msg_ref[...] = _pad128(msg, EB)


def _tc_msg(ea, hs, we1, be1_2, rsel, ssel, w2p, be2m):
    return pl.pallas_call(
        _msg_body,
        grid=(E // EB,),
        in_specs=[
            pl.BlockSpec((EB, EDGE_DIM), lambda i: (i, 0)),
            pl.BlockSpec((EB, W128), lambda i: (i, 0)),
            pl.BlockSpec((EDGE_DIM, H), lambda i: (0, 0)),
            pl.BlockSpec((1, H), lambda i: (0, 0)),
            pl.BlockSpec((H, HH), lambda i: (0, 0)),
            pl.BlockSpec((H, HH), lambda i: (0, 0)),
            pl.BlockSpec((HH, H), lambda i: (0, 0)),
            pl.BlockSpec((H, H), lambda i: (0, 0)),
        ],
        out_specs=pl.BlockSpec((EB, W128), lambda i: (i, 0)),
        out_shape=jax.ShapeDtypeStruct((E, W128), jnp.float32),
    )(ea, hs, we1, be1_2, rsel, ssel, w2p, be2m)


def _upd_body(h_ref, part_ref, inv_ref, wr_ref, bconv_ref,
              wir_ref, wiz_ref, win_ref, whr_ref, whz_ref, whn_ref,
              bir_ref, biz_ref, bin_ref, bhr_ref, bhz_ref, bhn_ref,
              hout_ref):
    h = h_ref[:, :H]
    agg = (part_ref[0, :, :H] + part_ref[1, :, :H]) * inv_ref[...]
    m = jnp.maximum(agg + _dot(h, wr_ref[...]) + bconv_ref[...], 0.0)
    r = jax.nn.sigmoid(_dot(m, wir_ref[...]) + bir_ref[...]
                       + _dot(h, whr_ref[...]) + bhr_ref[...])
    zz = jax.nn.sigmoid(_dot(m, wiz_ref[...]) + biz_ref[...]
                        + _dot(h, whz_ref[...]) + bhz_ref[...])
    hn = _dot(h, whn_ref[...]) + bhn_ref[...]
    cand = jnp.tanh(_dot(m, win_ref[...]) + bin_ref[...] + r * hn)
    hout_ref[...] = _pad128((1.0 - zz) * cand + zz * h, NB)


def _tc_update(h, parts, inv32, wr, bconv2, gru_ws, gru_bs):
    wspec = pl.BlockSpec((H, H), lambda i: (0, 0))
    bspec = pl.BlockSpec((1, H), lambda i: (0, 0))
    return pl.pallas_call(
        _upd_body,
        grid=(N // NB,),
        in_specs=[
            pl.BlockSpec((NB, W128), lambda i: (i, 0)),
            pl.BlockSpec((NC, NB, W128), lambda i: (0, i, 0)),
            pl.BlockSpec((NB, H), lambda i: (i, 0)),
            wspec, bspec,
            wspec, wspec, wspec, wspec, wspec, wspec,
            bspec, bspec, bspec, bspec, bspec, bspec,
        ],
        out_specs=pl.BlockSpec((NB, W128), lambda i: (i, 0)),
        out_shape=jax.ShapeDtypeStruct((N, W128), jnp.float32),
    )(h, parts, inv32, wr, bconv2, *gru_ws, *gru_bs)


def _mlp_body(hs_ref, hd_ref, ea_ref, wa_ref, wb_ref, wc_ref, b1_ref,
              w2_ref, b2_ref, w3_ref, b3_ref, out_ref):
    d1 = jnp.maximum(_dot(hs_ref[:, :H], wa_ref[...])
                     + _dot(hd_ref[:, :H], wb_ref[...])
                     + _dot(ea_ref[...], wc_ref[...]) + b1_ref[...], 0.0)
    d2 = jnp.maximum(_dot(d1, w2_ref[...]) + b2_ref[...], 0.0)
    out_ref[...] = _dot(d2, w3_ref[...]) + b3_ref[...]


def _tc_mlp(hs, hd, ea, wa, wb, wc, b1_2, w2, b2_2, w3, b3_2, num_targets):
    return pl.pallas_call(
        _mlp_body,
        grid=(E // EB2,),
        in_specs=[
            pl.BlockSpec((EB2, W128), lambda i: (i, 0)),
            pl.BlockSpec((EB2, W128), lambda i: (i, 0)),
            pl.BlockSpec((EB2, EDGE_DIM), lambda i: (i, 0)),
            pl.BlockSpec((H, H), lambda i: (0, 0)),
            pl.BlockSpec((H, H), lambda i: (0, 0)),
            pl.BlockSpec((EDGE_DIM, H), lambda i: (0, 0)),
            pl.BlockSpec((1, H), lambda i: (0, 0)),
            pl.BlockSpec((H, H // 2), lambda i: (0, 0)),
            pl.BlockSpec((1, H // 2), lambda i: (0, 0)),
            pl.BlockSpec((H // 2, num_targets), lambda i: (0, 0)),
            pl.BlockSpec((1, num_targets), lambda i: (0, 0)),
        ],
        out_specs=pl.BlockSpec((EB2, num_targets), lambda i: (i, 0)),
        out_shape=jax.ShapeDtypeStruct((E, num_targets), jnp.float32),
    )(hs, hd, ea, wa, wb, wc, b1_2, w2, b2_2, w3, b3_2)


def kernel(x, edge_index, edge_attr, u, W_in, b_in, We1, be1, We2, be2,
           W_root, b_conv, W_ih, W_hh, b_ih, b_hh,
           Wd1, bd1, Wd2, bd2, Wd3, bd3):
    num_targets = Wd3.shape[1]
    src = edge_index[0]
    dst = edge_index[1]

    w2p = We2.reshape(HH, H)
    be2m = be2.reshape(H, H)
    eye = jnp.eye(H, dtype=jnp.float32)
    rsel = jnp.repeat(eye, H, axis=1)
    ssel = jnp.tile(eye, (1, H))
    wx, wu = W_in[:NODE_DIM], W_in[NODE_DIM:]
    b_in2 = b_in.reshape(1, H)
    bconv2 = b_conv.reshape(1, H)
    wihT = W_ih.T
    whhT = W_hh.T
    gru_ws = (wihT[:, :H], wihT[:, H:2 * H], wihT[:, 2 * H:],
              whhT[:, :H], whhT[:, H:2 * H], whhT[:, 2 * H:])
    gru_bs = (b_ih[:H].reshape(1, H), b_ih[H:2 * H].reshape(1, H),
              b_ih[2 * H:].reshape(1, H), b_hh[:H].reshape(1, H),
              b_hh[H:2 * H].reshape(1, H), b_hh[2 * H:].reshape(1, H))
    wr = W_root
    be1_2 = be1.reshape(1, H)
    ones_cw = jnp.ones((CH, W128), jnp.float32)
    zeros_nw = jnp.zeros((N, W128), jnp.float32)
    wa, wb, wc = Wd1[:H], Wd1[H:2 * H], Wd1[2 * H:]

    cnt_parts = _sc_counts(dst, ones_cw, zeros_nw)

    h, inv32 = _tc_prep(x, u, wx, wu, b_in2, cnt_parts)

    for _ in range(3):
        hs = _sc_gather(h, src)
        msg = _tc_msg(edge_attr, hs, We1, be1_2, rsel, ssel, w2p, be2m)
        parts = _sc_scatter(msg, dst, zeros_nw)
        h = _tc_update(h, parts, inv32, wr, bconv2, gru_ws, gru_bs)

    hs = _sc_gather(h, src)
    hd = _sc_gather(h, dst)
    return _tc_mlp(hs, hd, edge_attr, wa, wb, wc, bd1.reshape(1, H),
                   Wd2, bd2.reshape(1, H // 2), Wd3,
                   bd3.reshape(1, num_targets), num_targets)


# speedup vs baseline: 1.9804x; 1.0234x over previous
import jax
import jax.numpy as jnp
from jax import lax
from jax.experimental import pallas as pl
from jax.experimental.pallas import tpu as pltpu
from jax.experimental.pallas import tpu_sc as plsc

N = 10000
E = 160000
H = 32
HH = H * H
W128 = 128
NODE_DIM = 12
GLOBAL_DIM = 11
EDGE_DIM = 5

NC, NS = 2, 16
NW = NC * NS
CH = 128
NCHUNK = E // CH
CPW = NCHUNK // NW
CEXTRA = NCHUNK - CPW * NW
STRIPE = 632

EB = 1000
EB2 = 2000
NB = 1000

_MESH = plsc.VectorSubcoreMesh(
    core_axis_name="c", subcore_axis_name="s", num_cores=NC, num_subcores=NS)


def _dot(a, b):
    return jnp.dot(a, b, preferred_element_type=jnp.float32)


def _pad128(v, rows):
    return jnp.concatenate(
        [v, jnp.zeros((rows, W128 - H), jnp.float32)], axis=1)


def _pad128_count(v, rows):
    return jnp.concatenate(
        [v, jnp.ones((rows, 1), jnp.float32),
         jnp.zeros((rows, W128 - H - 1), jnp.float32)], axis=1)


def _gather_body(table_hbm, idx_hbm, out_hbm, idx_v, rows_v, sem):
    c = lax.axis_index("c")
    s = lax.axis_index("s")
    w = c * NS + s

    @pl.loop(0, CPW)
    def _chunk(j):
        off = (j * NW + w) * CH
        pltpu.sync_copy(idx_hbm.at[pl.ds(off, CH)], idx_v)
        pltpu.async_copy(table_hbm.at[idx_v], rows_v, sem).wait()
        pltpu.sync_copy(rows_v, out_hbm.at[pl.ds(off, CH)])

    @pl.when(w < CEXTRA)
    def _extra():
        off = (CPW * NW + w) * CH
        pltpu.sync_copy(idx_hbm.at[pl.ds(off, CH)], idx_v)
        pltpu.async_copy(table_hbm.at[idx_v], rows_v, sem).wait()
        pltpu.sync_copy(rows_v, out_hbm.at[pl.ds(off, CH)])


def _sc_gather(table, idx):
    return pl.kernel(
        _gather_body,
        out_type=jax.ShapeDtypeStruct((E, W128), jnp.float32),
        mesh=_MESH,
        scratch_types=[
            pltpu.VMEM((CH,), jnp.int32),
            pltpu.VMEM((CH, W128), jnp.float32),
            pltpu.SemaphoreType.DMA,
        ],
    )(table, idx)


def _scatter_body(msg_hbm, idx_hbm, zeros_hbm, out_hbm, idx_v, rows_v, acc_sh):
    c = lax.axis_index("c")
    s = lax.axis_index("s")
    w = c * NS + s
    stripe_off = jnp.minimum(s * STRIPE, N - STRIPE)
    pltpu.sync_copy(zeros_hbm.at[pl.ds(stripe_off, STRIPE)],
                    acc_sh.at[pl.ds(stripe_off, STRIPE)])
    plsc.subcore_barrier()

    @pl.loop(0, CPW)
    def _chunk(j):
        off = (j * NW + w) * CH
        pltpu.sync_copy(idx_hbm.at[pl.ds(off, CH)], idx_v)
        pltpu.sync_copy(msg_hbm.at[pl.ds(off, CH)], rows_v)
        pltpu.sync_copy(rows_v, acc_sh.at[idx_v], add=True)

    @pl.when(w < CEXTRA)
    def _extra():
        off = (CPW * NW + w) * CH
        pltpu.sync_copy(idx_hbm.at[pl.ds(off, CH)], idx_v)
        pltpu.sync_copy(msg_hbm.at[pl.ds(off, CH)], rows_v)
        pltpu.sync_copy(rows_v, acc_sh.at[idx_v], add=True)

    plsc.subcore_barrier()
    pltpu.sync_copy(acc_sh.at[pl.ds(stripe_off, STRIPE)],
                    out_hbm.at[c].at[pl.ds(stripe_off, STRIPE)])


def _sc_scatter(msg, idx, zeros_nw):
    return pl.kernel(
        _scatter_body,
        out_type=jax.ShapeDtypeStruct((NC, N, W128), jnp.float32),
        mesh=_MESH,
        scratch_types=[
            pltpu.VMEM((CH,), jnp.int32),
            pltpu.VMEM((CH, W128), jnp.float32),
            pltpu.VMEM_SHARED((N, W128), jnp.float32),
        ],
    )(msg, idx, zeros_nw)


def _gather2_body(table_hbm, idx1_hbm, idx2_hbm, out1_hbm, out2_hbm,
                  idx_v, rows_v, sem):
    c = lax.axis_index("c")
    s = lax.axis_index("s")
    w = c * NS + s

    for idx_hbm, out_hbm in ((idx1_hbm, out1_hbm), (idx2_hbm, out2_hbm)):
        @pl.loop(0, CPW)
        def _chunk(j):
            off = (j * NW + w) * CH
            pltpu.sync_copy(idx_hbm.at[pl.ds(off, CH)], idx_v)
            pltpu.async_copy(table_hbm.at[idx_v], rows_v, sem).wait()
            pltpu.sync_copy(rows_v, out_hbm.at[pl.ds(off, CH)])

        @pl.when(w < CEXTRA)
        def _extra():
            off = (CPW * NW + w) * CH
            pltpu.sync_copy(idx_hbm.at[pl.ds(off, CH)], idx_v)
            pltpu.async_copy(table_hbm.at[idx_v], rows_v, sem).wait()
            pltpu.sync_copy(rows_v, out_hbm.at[pl.ds(off, CH)])


def _sc_gather2(table, idx1, idx2):
    return pl.kernel(
        _gather2_body,
        out_type=(jax.ShapeDtypeStruct((E, W128), jnp.float32),
                  jax.ShapeDtypeStruct((E, W128), jnp.float32)),
        mesh=_MESH,
        scratch_types=[
            pltpu.VMEM((CH,), jnp.int32),
            pltpu.VMEM((CH, W128), jnp.float32),
            pltpu.SemaphoreType.DMA,
        ],
    )(table, idx1, idx2)


def _prep_body(x_ref, u_ref, wx_ref, wu_ref, b_ref, h0_ref):
    ub = _dot(u_ref[...], wu_ref[...]) + b_ref[...]
    h0 = jnp.tanh(_dot(x_ref[...], wx_ref[...]) + ub)
    h0_ref[...] = _pad128(h0, NB)


def _tc_prep(x, u, wx, wu, b_in2):
    return pl.pallas_call(
        _prep_body,
        grid=(N // NB,),
        in_specs=[
            pl.BlockSpec((NB, NODE_DIM), lambda i: (i, 0)),
            pl.BlockSpec((1, GLOBAL_DIM), lambda i: (0, 0)),
            pl.BlockSpec((NODE_DIM, H), lambda i: (0, 0)),
            pl.BlockSpec((GLOBAL_DIM, H), lambda i: (0, 0)),
            pl.BlockSpec((1, H), lambda i: (0, 0)),
        ],
        out_specs=pl.BlockSpec((NB, W128), lambda i: (i, 0)),
        out_shape=jax.ShapeDtypeStruct((N, W128), jnp.float32),
    )(x, u, wx, wu, b_in2)


def _msg_body(ea_ref, hs_ref, we1_ref, be1_ref, rsel_ref, ssel_ref,
              w2p_ref, be2m_ref, msg_ref):
    z = jnp.maximum(_dot(ea_ref[...], we1_ref[...]) + be1_ref[...], 0.0)
    hs = hs_ref[:, :H]
    a = _dot(z, rsel_ref[...])
    b = _dot(hs, ssel_ref[...])
    p = a * b
    msg = _dot(p, w2p_ref[...]) + _dot(hs, be2m_ref[...])
    msg_ref[...] = _pad128_count(msg, EB)


def _tc_msg(ea, hs, we1, be1_2, rsel, ssel, w2p, be2m):
    return pl.pallas_call(
        _msg_body,
        grid=(E // EB,),
        in_specs=[
            pl.BlockSpec((EB, EDGE_DIM), lambda i: (i, 0)),
            pl.BlockSpec((EB, W128), lambda i: (i, 0)),
            pl.BlockSpec((EDGE_DIM, H), lambda i: (0, 0)),
            pl.BlockSpec((1, H), lambda i: (0, 0)),
            pl.BlockSpec((H, HH), lambda i: (0, 0)),
            pl.BlockSpec((H, HH), lambda i: (0, 0)),
            pl.BlockSpec((HH, H), lambda i: (0, 0)),
            pl.BlockSpec((H, H), lambda i: (0, 0)),
        ],
        out_specs=pl.BlockSpec((EB, W128), lambda i: (i, 0)),
        out_shape=jax.ShapeDtypeStruct((E, W128), jnp.float32),
    )(ea, hs, we1, be1_2, rsel, ssel, w2p, be2m)


def _upd_body(h_ref, part_ref, wr_ref, bconv_ref,
              wir_ref, wiz_ref, win_ref, whr_ref, whz_ref, whn_ref,
              bir_ref, biz_ref, bin_ref, bhr_ref, bhz_ref, bhn_ref,
              hout_ref):
    h = h_ref[:, :H]
    cnt = jnp.maximum(part_ref[0, :, H:H + 1] + part_ref[1, :, H:H + 1], 1.0)
    agg = (part_ref[0, :, :H] + part_ref[1, :, :H]) * (1.0 / cnt)
    m = jnp.maximum(agg + _dot(h, wr_ref[...]) + bconv_ref[...], 0.0)
    r = jax.nn.sigmoid(_dot(m, wir_ref[...]) + bir_ref[...]
                       + _dot(h, whr_ref[...]) + bhr_ref[...])
    zz = jax.nn.sigmoid(_dot(m, wiz_ref[...]) + biz_ref[...]
                        + _dot(h, whz_ref[...]) + bhz_ref[...])
    hn = _dot(h, whn_ref[...]) + bhn_ref[...]
    cand = jnp.tanh(_dot(m, win_ref[...]) + bin_ref[...] + r * hn)
    hout_ref[...] = _pad128((1.0 - zz) * cand + zz * h, NB)


def _tc_update(h, parts, wr, bconv2, gru_ws, gru_bs):
    wspec = pl.BlockSpec((H, H), lambda i: (0, 0))
    bspec = pl.BlockSpec((1, H), lambda i: (0, 0))
    return pl.pallas_call(
        _upd_body,
        grid=(N // NB,),
        in_specs=[
            pl.BlockSpec((NB, W128), lambda i: (i, 0)),
            pl.BlockSpec((NC, NB, W128), lambda i: (0, i, 0)),
            wspec, bspec,
            wspec, wspec, wspec, wspec, wspec, wspec,
            bspec, bspec, bspec, bspec, bspec, bspec,
        ],
        out_specs=pl.BlockSpec((NB, W128), lambda i: (i, 0)),
        out_shape=jax.ShapeDtypeStruct((N, W128), jnp.float32),
    )(h, parts, wr, bconv2, *gru_ws, *gru_bs)


def _mlp_body(hs_ref, hd_ref, ea_ref, wa_ref, wb_ref, wc_ref, b1_ref,
              w2_ref, b2_ref, w3_ref, b3_ref, out_ref):
    d1 = jnp.maximum(_dot(hs_ref[:, :H], wa_ref[...])
                     + _dot(hd_ref[:, :H], wb_ref[...])
                     + _dot(ea_ref[...], wc_ref[...]) + b1_ref[...], 0.0)
    d2 = jnp.maximum(_dot(d1, w2_ref[...]) + b2_ref[...], 0.0)
    out_ref[...] = _dot(d2, w3_ref[...]) + b3_ref[...]


def _tc_mlp(hs, hd, ea, wa, wb, wc, b1_2, w2, b2_2, w3, b3_2, num_targets):
    return pl.pallas_call(
        _mlp_body,
        grid=(E // EB2,),
        in_specs=[
            pl.BlockSpec((EB2, W128), lambda i: (i, 0)),
            pl.BlockSpec((EB2, W128), lambda i: (i, 0)),
            pl.BlockSpec((EB2, EDGE_DIM), lambda i: (i, 0)),
            pl.BlockSpec((H, H), lambda i: (0, 0)),
            pl.BlockSpec((H, H), lambda i: (0, 0)),
            pl.BlockSpec((EDGE_DIM, H), lambda i: (0, 0)),
            pl.BlockSpec((1, H), lambda i: (0, 0)),
            pl.BlockSpec((H, H // 2), lambda i: (0, 0)),
            pl.BlockSpec((1, H // 2), lambda i: (0, 0)),
            pl.BlockSpec((H // 2, num_targets), lambda i: (0, 0)),
            pl.BlockSpec((1, num_targets), lambda i: (0, 0)),
        ],
        out_specs=pl.BlockSpec((EB2, num_targets), lambda i: (i, 0)),
        out_shape=jax.ShapeDtypeStruct((E, num_targets), jnp.float32),
    )(hs, hd, ea, wa, wb, wc, b1_2, w2, b2_2, w3, b3_2)


def kernel(x, edge_index, edge_attr, u, W_in, b_in, We1, be1, We2, be2,
           W_root, b_conv, W_ih, W_hh, b_ih, b_hh,
           Wd1, bd1, Wd2, bd2, Wd3, bd3):
    num_targets = Wd3.shape[1]
    src = edge_index[0]
    dst = edge_index[1]

    w2p = We2.reshape(HH, H)
    be2m = be2.reshape(H, H)
    eye = jnp.eye(H, dtype=jnp.float32)
    rsel = jnp.repeat(eye, H, axis=1)
    ssel = jnp.tile(eye, (1, H))
    wx, wu = W_in[:NODE_DIM], W_in[NODE_DIM:]
    b_in2 = b_in.reshape(1, H)
    bconv2 = b_conv.reshape(1, H)
    wihT = W_ih.T
    whhT = W_hh.T
    gru_ws = (wihT[:, :H], wihT[:, H:2 * H], wihT[:, 2 * H:],
              whhT[:, :H], whhT[:, H:2 * H], whhT[:, 2 * H:])
    gru_bs = (b_ih[:H].reshape(1, H), b_ih[H:2 * H].reshape(1, H),
              b_ih[2 * H:].reshape(1, H), b_hh[:H].reshape(1, H),
              b_hh[H:2 * H].reshape(1, H), b_hh[2 * H:].reshape(1, H))
    wr = W_root
    be1_2 = be1.reshape(1, H)
    zeros_nw = jnp.zeros((N, W128), jnp.float32)
    wa, wb, wc = Wd1[:H], Wd1[H:2 * H], Wd1[2 * H:]

    h = _tc_prep(x, u, wx, wu, b_in2)

    for _ in range(3):
        hs = _sc_gather(h, src)
        msg = _tc_msg(edge_attr, hs, We1, be1_2, rsel, ssel, w2p, be2m)
        parts = _sc_scatter(msg, dst, zeros_nw)
        h = _tc_update(h, parts, wr, bconv2, gru_ws, gru_bs)

    hs, hd = _sc_gather2(h, src, dst)
    return _tc_mlp(hs, hd, edge_attr, wa, wb, wc, bd1.reshape(1, H),
                   Wd2, bd2.reshape(1, H // 2), Wd3,
                   bd3.reshape(1, num_targets), num_targets)


# speedup vs baseline: 2.2982x; 1.1605x over previous
import jax
import jax.numpy as jnp
from jax import lax
from jax.experimental import pallas as pl
from jax.experimental.pallas import tpu as pltpu
from jax.experimental.pallas import tpu_sc as plsc

N = 10000
E = 160000
H = 32
HH = H * H
W128 = 128
NODE_DIM = 12
GLOBAL_DIM = 11
EDGE_DIM = 5

NC, NS = 2, 16
NW = NC * NS
CH = 128
NCHUNK = E // CH
CPW = NCHUNK // NW
CEXTRA = NCHUNK - CPW * NW
STRIPE = 632

EB = 1000
EB2 = 2000
NB = 1000

_MESH = plsc.VectorSubcoreMesh(
    core_axis_name="c", subcore_axis_name="s", num_cores=NC, num_subcores=NS)


def _dot(a, b):
    return jnp.dot(a, b, preferred_element_type=jnp.float32)


def _pad128(v, rows):
    return jnp.concatenate(
        [v, jnp.zeros((rows, W128 - H), jnp.float32)], axis=1)


def _pad128_count(v, rows):
    return jnp.concatenate(
        [v, jnp.ones((rows, 1), jnp.float32),
         jnp.zeros((rows, W128 - H - 1), jnp.float32)], axis=1)


RING = 4
NPIPE = (CPW // RING) * RING


def _gather_pipe(table_hbm, idx_hbm, out_hbm, idxs, rowss, semi, semg, semo, w):
    def off(j):
        return (j * NW + w) * CH

    for b in range(2):
        pltpu.async_copy(idx_hbm.at[pl.ds(off(b), CH)], idxs[b], semi[b])

    @pl.loop(0, NPIPE // RING)
    def _grp(g):
        for t in range(RING):
            b = t
            b2 = (t + 2) % RING
            j = g * RING + t
            pltpu.make_async_copy(
                idx_hbm.at[pl.ds(0, CH)], idxs[b], semi[b]).wait()

            @pl.when(j >= RING)
            def _():
                pltpu.make_async_copy(
                    rowss[b], out_hbm.at[pl.ds(0, CH)], semo[b]).wait()

            pltpu.async_copy(table_hbm.at[idxs[b]], rowss[b], semg[b])

            @pl.when(j >= 2)
            def _():
                pltpu.make_async_copy(
                    table_hbm.at[pl.ds(0, CH)], rowss[b2], semg[b2]).wait()

            @pl.when(j + 2 < NPIPE)
            def _():
                pltpu.async_copy(
                    idx_hbm.at[pl.ds(off(j + 2), CH)], idxs[b2], semi[b2])

            @pl.when(j >= 2)
            def _():
                pltpu.async_copy(
                    rowss[b2], out_hbm.at[pl.ds(off(j - 2), CH)], semo[b2])

    for j in (NPIPE - 2, NPIPE - 1):
        b = j % RING
        pltpu.make_async_copy(
            table_hbm.at[pl.ds(0, CH)], rowss[b], semg[b]).wait()
        pltpu.async_copy(rowss[b], out_hbm.at[pl.ds(off(j), CH)], semo[b])
    for b in range(RING):
        pltpu.make_async_copy(
            rowss[b], out_hbm.at[pl.ds(0, CH)], semo[b]).wait()
    for j in range(NPIPE, CPW):
        pltpu.sync_copy(idx_hbm.at[pl.ds(off(j), CH)], idxs[0])
        pltpu.async_copy(table_hbm.at[idxs[0]], rowss[0], semg[0]).wait()
        pltpu.sync_copy(rowss[0], out_hbm.at[pl.ds(off(j), CH)])

    @pl.when(w < CEXTRA)
    def _extra():
        o = (CPW * NW + w) * CH
        pltpu.sync_copy(idx_hbm.at[pl.ds(o, CH)], idxs[0])
        pltpu.async_copy(table_hbm.at[idxs[0]], rowss[0], semg[0]).wait()
        pltpu.sync_copy(rowss[0], out_hbm.at[pl.ds(o, CH)])


_GATHER_SCRATCH = (
    [pltpu.VMEM((CH,), jnp.int32) for _ in range(RING)]
    + [pltpu.VMEM((CH, W128), jnp.float32) for _ in range(RING)]
    + [pltpu.SemaphoreType.DMA for _ in range(3 * RING)]
)


def _gather_body(table_hbm, idx_hbm, out_hbm, *scr):
    idxs, rowss = scr[:RING], scr[RING:2 * RING]
    semi, semg, semo = (scr[2 * RING:3 * RING], scr[3 * RING:4 * RING],
                        scr[4 * RING:5 * RING])
    w = lax.axis_index("c") * NS + lax.axis_index("s")
    _gather_pipe(table_hbm, idx_hbm, out_hbm, idxs, rowss, semi, semg, semo, w)


def _sc_gather(table, idx):
    return pl.kernel(
        _gather_body,
        out_type=jax.ShapeDtypeStruct((E, W128), jnp.float32),
        mesh=_MESH,
        scratch_types=list(_GATHER_SCRATCH),
    )(table, idx)


SRING = 2
SPIPE = (CPW // SRING) * SRING


def _scatter_body(msg_hbm, idx_hbm, zeros_hbm, out_hbm, *scr):
    idxs, rowss = scr[:SRING], scr[SRING:2 * SRING]
    semi, semm, sems = (scr[2 * SRING:3 * SRING], scr[3 * SRING:4 * SRING],
                        scr[4 * SRING:5 * SRING])
    acc_sh = scr[5 * SRING]
    c = lax.axis_index("c")
    s = lax.axis_index("s")
    w = c * NS + s
    stripe_off = jnp.minimum(s * STRIPE, N - STRIPE)
    pltpu.sync_copy(zeros_hbm.at[pl.ds(stripe_off, STRIPE)],
                    acc_sh.at[pl.ds(stripe_off, STRIPE)])

    def off(j):
        return (j * NW + w) * CH

    plsc.subcore_barrier()
    pltpu.async_copy(idx_hbm.at[pl.ds(off(0), CH)], idxs[0], semi[0])
    pltpu.async_copy(msg_hbm.at[pl.ds(off(0), CH)], rowss[0], semm[0])

    @pl.loop(0, SPIPE // SRING)
    def _grp(g):
        for t in range(SRING):
            b = t
            b2 = t ^ 1
            j = g * SRING + t
            pltpu.make_async_copy(
                idx_hbm.at[pl.ds(0, CH)], idxs[b], semi[b]).wait()
            pltpu.make_async_copy(
                msg_hbm.at[pl.ds(0, CH)], rowss[b], semm[b]).wait()
            pltpu.async_copy(rowss[b], acc_sh.at[idxs[b]], sems[b], add=True)

            @pl.when(j >= 1)
            def _():
                pltpu.make_async_copy(
                    rowss[b2], acc_sh.at[pl.ds(0, CH)], sems[b2]).wait()

            @pl.when(j + 1 < SPIPE)
            def _():
                pltpu.async_copy(
                    idx_hbm.at[pl.ds(off(j + 1), CH)], idxs[b2], semi[b2])
                pltpu.async_copy(
                    msg_hbm.at[pl.ds(off(j + 1), CH)], rowss[b2], semm[b2])

    b_last = (SPIPE - 1) % SRING
    pltpu.make_async_copy(
        rowss[b_last], acc_sh.at[pl.ds(0, CH)], sems[b_last]).wait()
    for j in range(SPIPE, CPW):
        pltpu.sync_copy(idx_hbm.at[pl.ds(off(j), CH)], idxs[0])
        pltpu.sync_copy(msg_hbm.at[pl.ds(off(j), CH)], rowss[0])
        pltpu.sync_copy(rowss[0], acc_sh.at[idxs[0]], add=True)

    @pl.when(w < CEXTRA)
    def _extra():
        o = (CPW * NW + w) * CH
        pltpu.sync_copy(idx_hbm.at[pl.ds(o, CH)], idxs[0])
        pltpu.sync_copy(msg_hbm.at[pl.ds(o, CH)], rowss[0])
        pltpu.sync_copy(rowss[0], acc_sh.at[idxs[0]], add=True)

    plsc.subcore_barrier()
    pltpu.sync_copy(acc_sh.at[pl.ds(stripe_off, STRIPE)],
                    out_hbm.at[c].at[pl.ds(stripe_off, STRIPE)])


_SCATTER_SCRATCH = (
    [pltpu.VMEM((CH,), jnp.int32) for _ in range(SRING)]
    + [pltpu.VMEM((CH, W128), jnp.float32) for _ in range(SRING)]
    + [pltpu.SemaphoreType.DMA for _ in range(3 * SRING)]
    + [pltpu.VMEM_SHARED((N, W128), jnp.float32)]
)


def _sc_scatter(msg, idx, zeros_nw):
    return pl.kernel(
        _scatter_body,
        out_type=jax.ShapeDtypeStruct((NC, N, W128), jnp.float32),
        mesh=_MESH,
        scratch_types=list(_SCATTER_SCRATCH),
    )(msg, idx, zeros_nw)


def _gather2_body(table_hbm, idx1_hbm, idx2_hbm, out1_hbm, out2_hbm, *scr):
    idxs, rowss = scr[:RING], scr[RING:2 * RING]
    semi, semg, semo = (scr[2 * RING:3 * RING], scr[3 * RING:4 * RING],
                        scr[4 * RING:5 * RING])
    w = lax.axis_index("c") * NS + lax.axis_index("s")
    _gather_pipe(table_hbm, idx1_hbm, out1_hbm, idxs, rowss,
                 semi, semg, semo, w)
    _gather_pipe(table_hbm, idx2_hbm, out2_hbm, idxs, rowss,
                 semi, semg, semo, w)


def _sc_gather2(table, idx1, idx2):
    return pl.kernel(
        _gather2_body,
        out_type=(jax.ShapeDtypeStruct((E, W128), jnp.float32),
                  jax.ShapeDtypeStruct((E, W128), jnp.float32)),
        mesh=_MESH,
        scratch_types=list(_GATHER_SCRATCH),
    )(table, idx1, idx2)


def _prep_body(x_ref, u_ref, wx_ref, wu_ref, b_ref, h0_ref):
    ub = _dot(u_ref[...], wu_ref[...]) + b_ref[...]
    h0 = jnp.tanh(_dot(x_ref[...], wx_ref[...]) + ub)
    h0_ref[...] = _pad128(h0, NB)


def _tc_prep(x, u, wx, wu, b_in2):
    return pl.pallas_call(
        _prep_body,
        grid=(N // NB,),
        in_specs=[
            pl.BlockSpec((NB, NODE_DIM), lambda i: (i, 0)),
            pl.BlockSpec((1, GLOBAL_DIM), lambda i: (0, 0)),
            pl.BlockSpec((NODE_DIM, H), lambda i: (0, 0)),
            pl.BlockSpec((GLOBAL_DIM, H), lambda i: (0, 0)),
            pl.BlockSpec((1, H), lambda i: (0, 0)),
        ],
        out_specs=pl.BlockSpec((NB, W128), lambda i: (i, 0)),
        out_shape=jax.ShapeDtypeStruct((N, W128), jnp.float32),
    )(x, u, wx, wu, b_in2)


def _msg_body(ea_ref, hs_ref, we1_ref, be1_ref, rsel_ref, ssel_ref,
              w2p_ref, be2m_ref, msg_ref):
    z = jnp.maximum(_dot(ea_ref[...], we1_ref[...]) + be1_ref[...], 0.0)
    hs = hs_ref[:, :H]
    a = _dot(z, rsel_ref[...])
    b = _dot(hs, ssel_ref[...])
    p = a * b
    msg = _dot(p, w2p_ref[...]) + _dot(hs, be2m_ref[...])
    msg_ref[...] = _pad128_count(msg, EB)


def _tc_msg(ea, hs, we1, be1_2, rsel, ssel, w2p, be2m):
    return pl.pallas_call(
        _msg_body,
        grid=(E // EB,),
        in_specs=[
            pl.BlockSpec((EB, EDGE_DIM), lambda i: (i, 0)),
            pl.BlockSpec((EB, W128), lambda i: (i, 0)),
            pl.BlockSpec((EDGE_DIM, H), lambda i: (0, 0)),
            pl.BlockSpec((1, H), lambda i: (0, 0)),
            pl.BlockSpec((H, HH), lambda i: (0, 0)),
            pl.BlockSpec((H, HH), lambda i: (0, 0)),
            pl.BlockSpec((HH, H), lambda i: (0, 0)),
            pl.BlockSpec((H, H), lambda i: (0, 0)),
        ],
        out_specs=pl.BlockSpec((EB, W128), lambda i: (i, 0)),
        out_shape=jax.ShapeDtypeStruct((E, W128), jnp.float32),
    )(ea, hs, we1, be1_2, rsel, ssel, w2p, be2m)


def _upd_body(h_ref, part_ref, wr_ref, bconv_ref,
              wir_ref, wiz_ref, win_ref, whr_ref, whz_ref, whn_ref,
              bir_ref, biz_ref, bin_ref, bhr_ref, bhz_ref, bhn_ref,
              hout_ref):
    h = h_ref[:, :H]
    cnt = jnp.maximum(part_ref[0, :, H:H + 1] + part_ref[1, :, H:H + 1], 1.0)
    agg = (part_ref[0, :, :H] + part_ref[1, :, :H]) * (1.0 / cnt)
    m = jnp.maximum(agg + _dot(h, wr_ref[...]) + bconv_ref[...], 0.0)
    r = jax.nn.sigmoid(_dot(m, wir_ref[...]) + bir_ref[...]
                       + _dot(h, whr_ref[...]) + bhr_ref[...])
    zz = jax.nn.sigmoid(_dot(m, wiz_ref[...]) + biz_ref[...]
                        + _dot(h, whz_ref[...]) + bhz_ref[...])
    hn = _dot(h, whn_ref[...]) + bhn_ref[...]
    cand = jnp.tanh(_dot(m, win_ref[...]) + bin_ref[...] + r * hn)
    hout_ref[...] = _pad128((1.0 - zz) * cand + zz * h, NB)


def _tc_update(h, parts, wr, bconv2, gru_ws, gru_bs):
    wspec = pl.BlockSpec((H, H), lambda i: (0, 0))
    bspec = pl.BlockSpec((1, H), lambda i: (0, 0))
    return pl.pallas_call(
        _upd_body,
        grid=(N // NB,),
        in_specs=[
            pl.BlockSpec((NB, W128), lambda i: (i, 0)),
            pl.BlockSpec((NC, NB, W128), lambda i: (0, i, 0)),
            wspec, bspec,
            wspec, wspec, wspec, wspec, wspec, wspec,
            bspec, bspec, bspec, bspec, bspec, bspec,
        ],
        out_specs=pl.BlockSpec((NB, W128), lambda i: (i, 0)),
        out_shape=jax.ShapeDtypeStruct((N, W128), jnp.float32),
    )(h, parts, wr, bconv2, *gru_ws, *gru_bs)


def _mlp_body(hs_ref, hd_ref, ea_ref, wa_ref, wb_ref, wc_ref, b1_ref,
              w2_ref, b2_ref, w3_ref, b3_ref, out_ref):
    d1 = jnp.maximum(_dot(hs_ref[:, :H], wa_ref[...])
                     + _dot(hd_ref[:, :H], wb_ref[...])
                     + _dot(ea_ref[...], wc_ref[...]) + b1_ref[...], 0.0)
    d2 = jnp.maximum(_dot(d1, w2_ref[...]) + b2_ref[...], 0.0)
    out_ref[...] = _dot(d2, w3_ref[...]) + b3_ref[...]


def _tc_mlp(hs, hd, ea, wa, wb, wc, b1_2, w2, b2_2, w3, b3_2, num_targets):
    return pl.pallas_call(
        _mlp_body,
        grid=(E // EB2,),
        in_specs=[
            pl.BlockSpec((EB2, W128), lambda i: (i, 0)),
            pl.BlockSpec((EB2, W128), lambda i: (i, 0)),
            pl.BlockSpec((EB2, EDGE_DIM), lambda i: (i, 0)),
            pl.BlockSpec((H, H), lambda i: (0, 0)),
            pl.BlockSpec((H, H), lambda i: (0, 0)),
            pl.BlockSpec((EDGE_DIM, H), lambda i: (0, 0)),
            pl.BlockSpec((1, H), lambda i: (0, 0)),
            pl.BlockSpec((H, H // 2), lambda i: (0, 0)),
            pl.BlockSpec((1, H // 2), lambda i: (0, 0)),
            pl.BlockSpec((H // 2, num_targets), lambda i: (0, 0)),
            pl.BlockSpec((1, num_targets), lambda i: (0, 0)),
        ],
        out_specs=pl.BlockSpec((EB2, num_targets), lambda i: (i, 0)),
        out_shape=jax.ShapeDtypeStruct((E, num_targets), jnp.float32),
    )(hs, hd, ea, wa, wb, wc, b1_2, w2, b2_2, w3, b3_2)


def kernel(x, edge_index, edge_attr, u, W_in, b_in, We1, be1, We2, be2,
           W_root, b_conv, W_ih, W_hh, b_ih, b_hh,
           Wd1, bd1, Wd2, bd2, Wd3, bd3):
    num_targets = Wd3.shape[1]
    src = edge_index[0]
    dst = edge_index[1]

    w2p = We2.reshape(HH, H)
    be2m = be2.reshape(H, H)
    eye = jnp.eye(H, dtype=jnp.float32)
    rsel = jnp.repeat(eye, H, axis=1)
    ssel = jnp.tile(eye, (1, H))
    wx, wu = W_in[:NODE_DIM], W_in[NODE_DIM:]
    b_in2 = b_in.reshape(1, H)
    bconv2 = b_conv.reshape(1, H)
    wihT = W_ih.T
    whhT = W_hh.T
    gru_ws = (wihT[:, :H], wihT[:, H:2 * H], wihT[:, 2 * H:],
              whhT[:, :H], whhT[:, H:2 * H], whhT[:, 2 * H:])
    gru_bs = (b_ih[:H].reshape(1, H), b_ih[H:2 * H].reshape(1, H),
              b_ih[2 * H:].reshape(1, H), b_hh[:H].reshape(1, H),
              b_hh[H:2 * H].reshape(1, H), b_hh[2 * H:].reshape(1, H))
    wr = W_root
    be1_2 = be1.reshape(1, H)
    zeros_nw = jnp.zeros((N, W128), jnp.float32)
    wa, wb, wc = Wd1[:H], Wd1[H:2 * H], Wd1[2 * H:]

    h = _tc_prep(x, u, wx, wu, b_in2)

    for _ in range(3):
        hs = _sc_gather(h, src)
        msg = _tc_msg(edge_attr, hs, We1, be1_2, rsel, ssel, w2p, be2m)
        parts = _sc_scatter(msg, dst, zeros_nw)
        h = _tc_update(h, parts, wr, bconv2, gru_ws, gru_bs)

    hs, hd = _sc_gather2(h, src, dst)
    return _tc_mlp(hs, hd, edge_attr, wa, wb, wc, bd1.reshape(1, H),
                   Wd2, bd2.reshape(1, H // 2), Wd3,
                   bd3.reshape(1, num_targets), num_targets)


# speedup vs baseline: 2.9861x; 1.2993x over previous
import jax
import jax.numpy as jnp
from jax import lax
from jax.experimental import pallas as pl
from jax.experimental.pallas import tpu as pltpu
from jax.experimental.pallas import tpu_sc as plsc

N = 10000
E = 160000
H = 32
HH = H * H
W128 = 128
NODE_DIM = 12
GLOBAL_DIM = 11
EDGE_DIM = 5

NC, NS = 2, 16
NW = NC * NS
CH = 128
NCHUNK = E // CH
CPW = NCHUNK // NW
CEXTRA = NCHUNK - CPW * NW
STRIPE = 632

EB = 1000
EB2 = 2000
NB = 1000

_MESH = plsc.VectorSubcoreMesh(
    core_axis_name="c", subcore_axis_name="s", num_cores=NC, num_subcores=NS)


def _dot(a, b):
    return jnp.dot(a, b, preferred_element_type=jnp.float32)


def _rep4(v):
    return jnp.concatenate([v, v, v, v], axis=1)


def _pad128_count(v, rows):
    return jnp.concatenate(
        [v, jnp.ones((rows, 1), jnp.float32),
         jnp.zeros((rows, W128 - H - 1), jnp.float32)], axis=1)


RING = 4
NPIPE = (CPW // RING) * RING


def _gather_pipe(table_hbm, idx_hbm, out_hbm, idxs, rowss, semi, semg, semo, w):
    def off(j):
        return (j * NW + w) * CH

    for b in range(2):
        pltpu.async_copy(idx_hbm.at[pl.ds(off(b), CH)], idxs[b], semi[b])

    @pl.loop(0, NPIPE // RING)
    def _grp(g):
        for t in range(RING):
            b = t
            b2 = (t + 2) % RING
            j = g * RING + t
            pltpu.make_async_copy(
                idx_hbm.at[pl.ds(0, CH)], idxs[b], semi[b]).wait()

            @pl.when(j >= RING)
            def _():
                pltpu.make_async_copy(
                    rowss[b], out_hbm.at[pl.ds(0, CH)], semo[b]).wait()

            pltpu.async_copy(table_hbm.at[idxs[b]], rowss[b], semg[b])

            @pl.when(j >= 2)
            def _():
                pltpu.make_async_copy(
                    table_hbm.at[pl.ds(0, CH)], rowss[b2], semg[b2]).wait()

            @pl.when(j + 2 < NPIPE)
            def _():
                pltpu.async_copy(
                    idx_hbm.at[pl.ds(off(j + 2), CH)], idxs[b2], semi[b2])

            @pl.when(j >= 2)
            def _():
                pltpu.async_copy(
                    rowss[b2], out_hbm.at[pl.ds(off(j - 2), CH)], semo[b2])

    for j in (NPIPE - 2, NPIPE - 1):
        b = j % RING
        pltpu.make_async_copy(
            table_hbm.at[pl.ds(0, CH)], rowss[b], semg[b]).wait()
        pltpu.async_copy(rowss[b], out_hbm.at[pl.ds(off(j), CH)], semo[b])
    for b in range(RING):
        pltpu.make_async_copy(
            rowss[b], out_hbm.at[pl.ds(0, CH)], semo[b]).wait()
    for j in range(NPIPE, CPW):
        pltpu.sync_copy(idx_hbm.at[pl.ds(off(j), CH)], idxs[0])
        pltpu.async_copy(table_hbm.at[idxs[0]], rowss[0], semg[0]).wait()
        pltpu.sync_copy(rowss[0], out_hbm.at[pl.ds(off(j), CH)])

    @pl.when(w < CEXTRA)
    def _extra():
        o = (CPW * NW + w) * CH
        pltpu.sync_copy(idx_hbm.at[pl.ds(o, CH)], idxs[0])
        pltpu.async_copy(table_hbm.at[idxs[0]], rowss[0], semg[0]).wait()
        pltpu.sync_copy(rowss[0], out_hbm.at[pl.ds(o, CH)])


_GATHER_SCRATCH = (
    [pltpu.VMEM((CH,), jnp.int32) for _ in range(RING)]
    + [pltpu.VMEM((CH, W128), jnp.float32) for _ in range(RING)]
    + [pltpu.SemaphoreType.DMA for _ in range(3 * RING)]
)


def _gather_body(table_hbm, idx_hbm, out_hbm, *scr):
    idxs, rowss = scr[:RING], scr[RING:2 * RING]
    semi, semg, semo = (scr[2 * RING:3 * RING], scr[3 * RING:4 * RING],
                        scr[4 * RING:5 * RING])
    w = lax.axis_index("c") * NS + lax.axis_index("s")
    _gather_pipe(table_hbm, idx_hbm, out_hbm, idxs, rowss, semi, semg, semo, w)


def _sc_gather(table, idx):
    return pl.kernel(
        _gather_body,
        out_type=jax.ShapeDtypeStruct((E, W128), jnp.float32),
        mesh=_MESH,
        scratch_types=list(_GATHER_SCRATCH),
    )(table, idx)


SRING = 2
SPIPE = (CPW // SRING) * SRING


def _scatter_body(msg_hbm, idx_hbm, zeros_hbm, out_hbm, *scr):
    idxs, rowss = scr[:SRING], scr[SRING:2 * SRING]
    semi, semm, sems = (scr[2 * SRING:3 * SRING], scr[3 * SRING:4 * SRING],
                        scr[4 * SRING:5 * SRING])
    acc_sh = scr[5 * SRING]
    c = lax.axis_index("c")
    s = lax.axis_index("s")
    w = c * NS + s
    stripe_off = jnp.minimum(s * STRIPE, N - STRIPE)
    pltpu.sync_copy(zeros_hbm.at[pl.ds(stripe_off, STRIPE)],
                    acc_sh.at[pl.ds(stripe_off, STRIPE)])

    def off(j):
        return (j * NW + w) * CH

    plsc.subcore_barrier()
    pltpu.async_copy(idx_hbm.at[pl.ds(off(0), CH)], idxs[0], semi[0])
    pltpu.async_copy(msg_hbm.at[pl.ds(off(0), CH)], rowss[0], semm[0])

    @pl.loop(0, SPIPE // SRING)
    def _grp(g):
        for t in range(SRING):
            b = t
            b2 = t ^ 1
            j = g * SRING + t
            pltpu.make_async_copy(
                idx_hbm.at[pl.ds(0, CH)], idxs[b], semi[b]).wait()
            pltpu.make_async_copy(
                msg_hbm.at[pl.ds(0, CH)], rowss[b], semm[b]).wait()
            pltpu.async_copy(rowss[b], acc_sh.at[idxs[b]], sems[b], add=True)

            @pl.when(j >= 1)
            def _():
                pltpu.make_async_copy(
                    rowss[b2], acc_sh.at[pl.ds(0, CH)], sems[b2]).wait()

            @pl.when(j + 1 < SPIPE)
            def _():
                pltpu.async_copy(
                    idx_hbm.at[pl.ds(off(j + 1), CH)], idxs[b2], semi[b2])
                pltpu.async_copy(
                    msg_hbm.at[pl.ds(off(j + 1), CH)], rowss[b2], semm[b2])

    b_last = (SPIPE - 1) % SRING
    pltpu.make_async_copy(
        rowss[b_last], acc_sh.at[pl.ds(0, CH)], sems[b_last]).wait()
    for j in range(SPIPE, CPW):
        pltpu.sync_copy(idx_hbm.at[pl.ds(off(j), CH)], idxs[0])
        pltpu.sync_copy(msg_hbm.at[pl.ds(off(j), CH)], rowss[0])
        pltpu.sync_copy(rowss[0], acc_sh.at[idxs[0]], add=True)

    @pl.when(w < CEXTRA)
    def _extra():
        o = (CPW * NW + w) * CH
        pltpu.sync_copy(idx_hbm.at[pl.ds(o, CH)], idxs[0])
        pltpu.sync_copy(msg_hbm.at[pl.ds(o, CH)], rowss[0])
        pltpu.sync_copy(rowss[0], acc_sh.at[idxs[0]], add=True)

    plsc.subcore_barrier()
    pltpu.sync_copy(acc_sh.at[pl.ds(stripe_off, STRIPE)],
                    out_hbm.at[c].at[pl.ds(stripe_off, STRIPE)])


_SCATTER_SCRATCH = (
    [pltpu.VMEM((CH,), jnp.int32) for _ in range(SRING)]
    + [pltpu.VMEM((CH, W128), jnp.float32) for _ in range(SRING)]
    + [pltpu.SemaphoreType.DMA for _ in range(3 * SRING)]
    + [pltpu.VMEM_SHARED((N, W128), jnp.float32)]
)


def _sc_scatter(msg, idx, zeros_nw):
    return pl.kernel(
        _scatter_body,
        out_type=jax.ShapeDtypeStruct((NC, N, W128), jnp.float32),
        mesh=_MESH,
        scratch_types=list(_SCATTER_SCRATCH),
    )(msg, idx, zeros_nw)


def _gather2_body(table_hbm, idx1_hbm, idx2_hbm, out1_hbm, out2_hbm, *scr):
    idxs, rowss = scr[:RING], scr[RING:2 * RING]
    semi, semg, semo = (scr[2 * RING:3 * RING], scr[3 * RING:4 * RING],
                        scr[4 * RING:5 * RING])
    w = lax.axis_index("c") * NS + lax.axis_index("s")
    _gather_pipe(table_hbm, idx1_hbm, out1_hbm, idxs, rowss,
                 semi, semg, semo, w)
    _gather_pipe(table_hbm, idx2_hbm, out2_hbm, idxs, rowss,
                 semi, semg, semo, w)


def _sc_gather2(table, idx1, idx2):
    return pl.kernel(
        _gather2_body,
        out_type=(jax.ShapeDtypeStruct((E, W128), jnp.float32),
                  jax.ShapeDtypeStruct((E, W128), jnp.float32)),
        mesh=_MESH,
        scratch_types=list(_GATHER_SCRATCH),
    )(table, idx1, idx2)


def _prep_body(x_ref, u_ref, wx_ref, wu_ref, b_ref, h0_ref):
    ub = _dot(u_ref[...], wu_ref[...]) + b_ref[...]
    h0 = jnp.tanh(_dot(x_ref[...], wx_ref[...]) + ub)
    h0_ref[...] = _rep4(h0)


def _tc_prep(x, u, wx, wu, b_in2):
    return pl.pallas_call(
        _prep_body,
        grid=(N // NB,),
        in_specs=[
            pl.BlockSpec((NB, NODE_DIM), lambda i: (i, 0)),
            pl.BlockSpec((1, GLOBAL_DIM), lambda i: (0, 0)),
            pl.BlockSpec((NODE_DIM, H), lambda i: (0, 0)),
            pl.BlockSpec((GLOBAL_DIM, H), lambda i: (0, 0)),
            pl.BlockSpec((1, H), lambda i: (0, 0)),
        ],
        out_specs=pl.BlockSpec((NB, W128), lambda i: (i, 0)),
        out_shape=jax.ShapeDtypeStruct((N, W128), jnp.float32),
    )(x, u, wx, wu, b_in2)


def _msg_body(ea_ref, hs_ref, we1rep_ref, be1rep_ref, w2p_ref, be2m_ref,
              msg_ref):
    hs = hs_ref[:, :H]
    hs4 = hs_ref[...]
    a = jnp.maximum(_dot(ea_ref[...], we1rep_ref[...]) + be1rep_ref[...], 0.0)
    p = jnp.concatenate(
        [(a[:, i * W128:(i + 1) * W128] * hs4).astype(jnp.bfloat16)
         for i in range(HH // W128)], axis=1)
    msg = (jnp.dot(p, w2p_ref[...], preferred_element_type=jnp.float32)
           + _dot(hs, be2m_ref[...]))
    msg_ref[...] = _pad128_count(msg, EB)


def _tc_msg(ea, hs, we1rep, be1rep, w2p_bf, be2m):
    return pl.pallas_call(
        _msg_body,
        grid=(E // EB,),
        in_specs=[
            pl.BlockSpec((EB, EDGE_DIM), lambda i: (i, 0)),
            pl.BlockSpec((EB, W128), lambda i: (i, 0)),
            pl.BlockSpec((EDGE_DIM, HH), lambda i: (0, 0)),
            pl.BlockSpec((1, HH), lambda i: (0, 0)),
            pl.BlockSpec((HH, H), lambda i: (0, 0)),
            pl.BlockSpec((H, H), lambda i: (0, 0)),
        ],
        out_specs=pl.BlockSpec((EB, W128), lambda i: (i, 0)),
        out_shape=jax.ShapeDtypeStruct((E, W128), jnp.float32),
    )(ea, hs, we1rep, be1rep, w2p_bf, be2m)


def _upd_body(h_ref, part_ref, wr_ref, bconv_ref,
              wir_ref, wiz_ref, win_ref, whr_ref, whz_ref, whn_ref,
              bir_ref, biz_ref, bin_ref, bhr_ref, bhz_ref, bhn_ref,
              hout_ref):
    h = h_ref[:, :H]
    cnt = jnp.maximum(part_ref[0, :, H:H + 1] + part_ref[1, :, H:H + 1], 1.0)
    agg = (part_ref[0, :, :H] + part_ref[1, :, :H]) * (1.0 / cnt)
    m = jnp.maximum(agg + _dot(h, wr_ref[...]) + bconv_ref[...], 0.0)
    r = jax.nn.sigmoid(_dot(m, wir_ref[...]) + bir_ref[...]
                       + _dot(h, whr_ref[...]) + bhr_ref[...])
    zz = jax.nn.sigmoid(_dot(m, wiz_ref[...]) + biz_ref[...]
                        + _dot(h, whz_ref[...]) + bhz_ref[...])
    hn = _dot(h, whn_ref[...]) + bhn_ref[...]
    cand = jnp.tanh(_dot(m, win_ref[...]) + bin_ref[...] + r * hn)
    hout_ref[...] = _rep4((1.0 - zz) * cand + zz * h)


def _tc_update(h, parts, wr, bconv2, gru_ws, gru_bs):
    wspec = pl.BlockSpec((H, H), lambda i: (0, 0))
    bspec = pl.BlockSpec((1, H), lambda i: (0, 0))
    return pl.pallas_call(
        _upd_body,
        grid=(N // NB,),
        in_specs=[
            pl.BlockSpec((NB, W128), lambda i: (i, 0)),
            pl.BlockSpec((NC, NB, W128), lambda i: (0, i, 0)),
            wspec, bspec,
            wspec, wspec, wspec, wspec, wspec, wspec,
            bspec, bspec, bspec, bspec, bspec, bspec,
        ],
        out_specs=pl.BlockSpec((NB, W128), lambda i: (i, 0)),
        out_shape=jax.ShapeDtypeStruct((N, W128), jnp.float32),
    )(h, parts, wr, bconv2, *gru_ws, *gru_bs)


def _mlp_body(hs_ref, hd_ref, ea_ref, wa_ref, wb_ref, wc_ref, b1_ref,
              w2_ref, b2_ref, w3_ref, b3_ref, out_ref):
    d1 = jnp.maximum(_dot(hs_ref[:, :H], wa_ref[...])
                     + _dot(hd_ref[:, :H], wb_ref[...])
                     + _dot(ea_ref[...], wc_ref[...]) + b1_ref[...], 0.0)
    d2 = jnp.maximum(_dot(d1, w2_ref[...]) + b2_ref[...], 0.0)
    out_ref[...] = _dot(d2, w3_ref[...]) + b3_ref[...]


def _tc_mlp(hs, hd, ea, wa, wb, wc, b1_2, w2, b2_2, w3, b3_2, num_targets):
    return pl.pallas_call(
        _mlp_body,
        grid=(E // EB2,),
        in_specs=[
            pl.BlockSpec((EB2, W128), lambda i: (i, 0)),
            pl.BlockSpec((EB2, W128), lambda i: (i, 0)),
            pl.BlockSpec((EB2, EDGE_DIM), lambda i: (i, 0)),
            pl.BlockSpec((H, H), lambda i: (0, 0)),
            pl.BlockSpec((H, H), lambda i: (0, 0)),
            pl.BlockSpec((EDGE_DIM, H), lambda i: (0, 0)),
            pl.BlockSpec((1, H), lambda i: (0, 0)),
            pl.BlockSpec((H, H // 2), lambda i: (0, 0)),
            pl.BlockSpec((1, H // 2), lambda i: (0, 0)),
            pl.BlockSpec((H // 2, num_targets), lambda i: (0, 0)),
            pl.BlockSpec((1, num_targets), lambda i: (0, 0)),
        ],
        out_specs=pl.BlockSpec((EB2, num_targets), lambda i: (i, 0)),
        out_shape=jax.ShapeDtypeStruct((E, num_targets), jnp.float32),
    )(hs, hd, ea, wa, wb, wc, b1_2, w2, b2_2, w3, b3_2)


def kernel(x, edge_index, edge_attr, u, W_in, b_in, We1, be1, We2, be2,
           W_root, b_conv, W_ih, W_hh, b_ih, b_hh,
           Wd1, bd1, Wd2, bd2, Wd3, bd3):
    num_targets = Wd3.shape[1]
    src = edge_index[0]
    dst = edge_index[1]

    w2p = We2.reshape(HH, H)
    be2m = be2.reshape(H, H)
    we1rep = jnp.repeat(We1, H, axis=1)
    be1rep = jnp.repeat(be1, H).reshape(1, HH)
    wx, wu = W_in[:NODE_DIM], W_in[NODE_DIM:]
    b_in2 = b_in.reshape(1, H)
    bconv2 = b_conv.reshape(1, H)
    wihT = W_ih.T
    whhT = W_hh.T
    gru_ws = (wihT[:, :H], wihT[:, H:2 * H], wihT[:, 2 * H:],
              whhT[:, :H], whhT[:, H:2 * H], whhT[:, 2 * H:])
    gru_bs = (b_ih[:H].reshape(1, H), b_ih[H:2 * H].reshape(1, H),
              b_ih[2 * H:].reshape(1, H), b_hh[:H].reshape(1, H),
              b_hh[H:2 * H].reshape(1, H), b_hh[2 * H:].reshape(1, H))
    wr = W_root
    w2p_bf = w2p.astype(jnp.bfloat16)
    zeros_nw = jnp.zeros((N, W128), jnp.float32)
    wa, wb, wc = Wd1[:H], Wd1[H:2 * H], Wd1[2 * H:]

    h = _tc_prep(x, u, wx, wu, b_in2)

    for _ in range(3):
        hs = _sc_gather(h, src)
        msg = _tc_msg(edge_attr, hs, we1rep, be1rep, w2p_bf, be2m)
        parts = _sc_scatter(msg, dst, zeros_nw)
        h = _tc_update(h, parts, wr, bconv2, gru_ws, gru_bs)

    hs, hd = _sc_gather2(h, src, dst)
    return _tc_mlp(hs, hd, edge_attr, wa, wb, wc, bd1.reshape(1, H),
                   Wd2, bd2.reshape(1, H // 2), Wd3,
                   bd3.reshape(1, num_targets), num_targets)


# speedup vs baseline: 3.1977x; 1.0709x over previous
import jax
import jax.numpy as jnp
from jax import lax
from jax.experimental import pallas as pl
from jax.experimental.pallas import tpu as pltpu
from jax.experimental.pallas import tpu_sc as plsc

N = 10000
E = 160000
H = 32
HH = H * H
W128 = 128
NODE_DIM = 12
GLOBAL_DIM = 11
EDGE_DIM = 5

NC, NS = 2, 16
NW = NC * NS
CH = 128
NCHUNK = E // CH
CPW = NCHUNK // NW
CEXTRA = NCHUNK - CPW * NW
STRIPE = 632

EB = 2000
EB2 = 2000
NB = 1000

_MESH = plsc.VectorSubcoreMesh(
    core_axis_name="c", subcore_axis_name="s", num_cores=NC, num_subcores=NS)


def _dot(a, b):
    return jnp.dot(a, b, preferred_element_type=jnp.float32)


def _rep4(v):
    return jnp.concatenate([v, v, v, v], axis=1)


def _pad128_count(v, rows):
    return jnp.concatenate(
        [v, jnp.ones((rows, 1), jnp.float32),
         jnp.zeros((rows, W128 - H - 1), jnp.float32)], axis=1)


RING = 4
NPIPE = (CPW // RING) * RING


def _gather_pipe(table_hbm, idx_hbm, out_hbm, idxs, rowss, semi, semg, semo, w):
    def off(j):
        return (j * NW + w) * CH

    for b in range(2):
        pltpu.async_copy(idx_hbm.at[pl.ds(off(b), CH)], idxs[b], semi[b])

    @pl.loop(0, NPIPE // RING)
    def _grp(g):
        for t in range(RING):
            b = t
            b2 = (t + 2) % RING
            j = g * RING + t
            pltpu.make_async_copy(
                idx_hbm.at[pl.ds(0, CH)], idxs[b], semi[b]).wait()

            @pl.when(j >= RING)
            def _():
                pltpu.make_async_copy(
                    rowss[b], out_hbm.at[pl.ds(0, CH)], semo[b]).wait()

            pltpu.async_copy(table_hbm.at[idxs[b]], rowss[b], semg[b])

            @pl.when(j >= 2)
            def _():
                pltpu.make_async_copy(
                    table_hbm.at[pl.ds(0, CH)], rowss[b2], semg[b2]).wait()

            @pl.when(j + 2 < NPIPE)
            def _():
                pltpu.async_copy(
                    idx_hbm.at[pl.ds(off(j + 2), CH)], idxs[b2], semi[b2])

            @pl.when(j >= 2)
            def _():
                pltpu.async_copy(
                    rowss[b2], out_hbm.at[pl.ds(off(j - 2), CH)], semo[b2])

    for j in (NPIPE - 2, NPIPE - 1):
        b = j % RING
        pltpu.make_async_copy(
            table_hbm.at[pl.ds(0, CH)], rowss[b], semg[b]).wait()
        pltpu.async_copy(rowss[b], out_hbm.at[pl.ds(off(j), CH)], semo[b])
    for b in range(RING):
        pltpu.make_async_copy(
            rowss[b], out_hbm.at[pl.ds(0, CH)], semo[b]).wait()
    for j in range(NPIPE, CPW):
        pltpu.sync_copy(idx_hbm.at[pl.ds(off(j), CH)], idxs[0])
        pltpu.async_copy(table_hbm.at[idxs[0]], rowss[0], semg[0]).wait()
        pltpu.sync_copy(rowss[0], out_hbm.at[pl.ds(off(j), CH)])

    @pl.when(w < CEXTRA)
    def _extra():
        o = (CPW * NW + w) * CH
        pltpu.sync_copy(idx_hbm.at[pl.ds(o, CH)], idxs[0])
        pltpu.async_copy(table_hbm.at[idxs[0]], rowss[0], semg[0]).wait()
        pltpu.sync_copy(rowss[0], out_hbm.at[pl.ds(o, CH)])


_GATHER_SCRATCH = (
    [pltpu.VMEM((CH,), jnp.int32) for _ in range(RING)]
    + [pltpu.VMEM((CH, W128), jnp.float32) for _ in range(RING)]
    + [pltpu.SemaphoreType.DMA for _ in range(3 * RING)]
)


def _gather_body(table_hbm, idx_hbm, out_hbm, *scr):
    idxs, rowss = scr[:RING], scr[RING:2 * RING]
    semi, semg, semo = (scr[2 * RING:3 * RING], scr[3 * RING:4 * RING],
                        scr[4 * RING:5 * RING])
    w = lax.axis_index("c") * NS + lax.axis_index("s")
    _gather_pipe(table_hbm, idx_hbm, out_hbm, idxs, rowss, semi, semg, semo, w)


def _sc_gather(table, idx):
    return pl.kernel(
        _gather_body,
        out_type=jax.ShapeDtypeStruct((E, W128), jnp.float32),
        mesh=_MESH,
        scratch_types=list(_GATHER_SCRATCH),
    )(table, idx)


SRING = 2
SPIPE = (CPW // SRING) * SRING


def _scatter_body(msg_hbm, idx_hbm, zeros_hbm, out_hbm, *scr):
    idxs, rowss = scr[:SRING], scr[SRING:2 * SRING]
    semi, semm, sems = (scr[2 * SRING:3 * SRING], scr[3 * SRING:4 * SRING],
                        scr[4 * SRING:5 * SRING])
    acc_sh = scr[5 * SRING]
    c = lax.axis_index("c")
    s = lax.axis_index("s")
    w = c * NS + s
    stripe_off = jnp.minimum(s * STRIPE, N - STRIPE)
    pltpu.sync_copy(zeros_hbm.at[pl.ds(stripe_off, STRIPE)],
                    acc_sh.at[pl.ds(stripe_off, STRIPE)])

    def off(j):
        return (j * NW + w) * CH

    plsc.subcore_barrier()
    pltpu.async_copy(idx_hbm.at[pl.ds(off(0), CH)], idxs[0], semi[0])
    pltpu.async_copy(msg_hbm.at[pl.ds(off(0), CH)], rowss[0], semm[0])

    @pl.loop(0, SPIPE // SRING)
    def _grp(g):
        for t in range(SRING):
            b = t
            b2 = t ^ 1
            j = g * SRING + t
            pltpu.make_async_copy(
                idx_hbm.at[pl.ds(0, CH)], idxs[b], semi[b]).wait()
            pltpu.make_async_copy(
                msg_hbm.at[pl.ds(0, CH)], rowss[b], semm[b]).wait()
            pltpu.async_copy(rowss[b], acc_sh.at[idxs[b]], sems[b], add=True)

            @pl.when(j >= 1)
            def _():
                pltpu.make_async_copy(
                    rowss[b2], acc_sh.at[pl.ds(0, CH)], sems[b2]).wait()

            @pl.when(j + 1 < SPIPE)
            def _():
                pltpu.async_copy(
                    idx_hbm.at[pl.ds(off(j + 1), CH)], idxs[b2], semi[b2])
                pltpu.async_copy(
                    msg_hbm.at[pl.ds(off(j + 1), CH)], rowss[b2], semm[b2])

    b_last = (SPIPE - 1) % SRING
    pltpu.make_async_copy(
        rowss[b_last], acc_sh.at[pl.ds(0, CH)], sems[b_last]).wait()
    for j in range(SPIPE, CPW):
        pltpu.sync_copy(idx_hbm.at[pl.ds(off(j), CH)], idxs[0])
        pltpu.sync_copy(msg_hbm.at[pl.ds(off(j), CH)], rowss[0])
        pltpu.sync_copy(rowss[0], acc_sh.at[idxs[0]], add=True)

    @pl.when(w < CEXTRA)
    def _extra():
        o = (CPW * NW + w) * CH
        pltpu.sync_copy(idx_hbm.at[pl.ds(o, CH)], idxs[0])
        pltpu.sync_copy(msg_hbm.at[pl.ds(o, CH)], rowss[0])
        pltpu.sync_copy(rowss[0], acc_sh.at[idxs[0]], add=True)

    plsc.subcore_barrier()
    pltpu.sync_copy(acc_sh.at[pl.ds(stripe_off, STRIPE)],
                    out_hbm.at[c].at[pl.ds(stripe_off, STRIPE)])


_SCATTER_SCRATCH = (
    [pltpu.VMEM((CH,), jnp.int32) for _ in range(SRING)]
    + [pltpu.VMEM((CH, W128), jnp.float32) for _ in range(SRING)]
    + [pltpu.SemaphoreType.DMA for _ in range(3 * SRING)]
    + [pltpu.VMEM_SHARED((N, W128), jnp.float32)]
)


def _sc_scatter(msg, idx, zeros_nw):
    return pl.kernel(
        _scatter_body,
        out_type=jax.ShapeDtypeStruct((NC, N, W128), jnp.float32),
        mesh=_MESH,
        scratch_types=list(_SCATTER_SCRATCH),
    )(msg, idx, zeros_nw)


def _gather2_body(table_hbm, idx1_hbm, idx2_hbm, out1_hbm, out2_hbm, *scr):
    idxs, rowss = scr[:RING], scr[RING:2 * RING]
    semi, semg, semo = (scr[2 * RING:3 * RING], scr[3 * RING:4 * RING],
                        scr[4 * RING:5 * RING])
    w = lax.axis_index("c") * NS + lax.axis_index("s")
    _gather_pipe(table_hbm, idx1_hbm, out1_hbm, idxs, rowss,
                 semi, semg, semo, w)
    _gather_pipe(table_hbm, idx2_hbm, out2_hbm, idxs, rowss,
                 semi, semg, semo, w)


def _sc_gather2(table, idx1, idx2):
    return pl.kernel(
        _gather2_body,
        out_type=(jax.ShapeDtypeStruct((E, W128), jnp.float32),
                  jax.ShapeDtypeStruct((E, W128), jnp.float32)),
        mesh=_MESH,
        scratch_types=list(_GATHER_SCRATCH),
    )(table, idx1, idx2)


def _prep_body(x_ref, u_ref, wx_ref, wu_ref, b_ref, h0_ref):
    ub = _dot(u_ref[...], wu_ref[...]) + b_ref[...]
    h0 = jnp.tanh(_dot(x_ref[...], wx_ref[...]) + ub)
    h0_ref[...] = _rep4(h0)


def _tc_prep(x, u, wx, wu, b_in2):
    return pl.pallas_call(
        _prep_body,
        grid=(N // NB,),
        in_specs=[
            pl.BlockSpec((NB, NODE_DIM), lambda i: (i, 0)),
            pl.BlockSpec((1, GLOBAL_DIM), lambda i: (0, 0)),
            pl.BlockSpec((NODE_DIM, H), lambda i: (0, 0)),
            pl.BlockSpec((GLOBAL_DIM, H), lambda i: (0, 0)),
            pl.BlockSpec((1, H), lambda i: (0, 0)),
        ],
        out_specs=pl.BlockSpec((NB, W128), lambda i: (i, 0)),
        out_shape=jax.ShapeDtypeStruct((N, W128), jnp.float32),
    )(x, u, wx, wu, b_in2)


def _msg_body(ea_ref, hs_ref, we1rep_ref, be1rep_ref, w2p_ref, be2m_ref,
              msg_ref):
    hs = hs_ref[:, :H]
    hs4 = hs_ref[...]
    a = jnp.maximum(_dot(ea_ref[...], we1rep_ref[...]) + be1rep_ref[...], 0.0)
    p = jnp.concatenate(
        [(a[:, i * W128:(i + 1) * W128] * hs4).astype(jnp.bfloat16)
         for i in range(HH // W128)], axis=1)
    msg = (jnp.dot(p, w2p_ref[...], preferred_element_type=jnp.float32)
           + _dot(hs, be2m_ref[...]))
    msg_ref[...] = _pad128_count(msg, EB)


def _tc_msg(ea, hs, we1rep, be1rep, w2p_bf, be2m):
    return pl.pallas_call(
        _msg_body,
        grid=(E // EB,),
        in_specs=[
            pl.BlockSpec((EB, EDGE_DIM), lambda i: (i, 0)),
            pl.BlockSpec((EB, W128), lambda i: (i, 0)),
            pl.BlockSpec((EDGE_DIM, HH), lambda i: (0, 0)),
            pl.BlockSpec((1, HH), lambda i: (0, 0)),
            pl.BlockSpec((HH, H), lambda i: (0, 0)),
            pl.BlockSpec((H, H), lambda i: (0, 0)),
        ],
        out_specs=pl.BlockSpec((EB, W128), lambda i: (i, 0)),
        out_shape=jax.ShapeDtypeStruct((E, W128), jnp.float32),
    )(ea, hs, we1rep, be1rep, w2p_bf, be2m)


def _upd_body(h_ref, part_ref, wr_ref, bconv_ref,
              wir_ref, wiz_ref, win_ref, whr_ref, whz_ref, whn_ref,
              bir_ref, biz_ref, bin_ref, bhr_ref, bhz_ref, bhn_ref,
              hout_ref):
    h = h_ref[:, :H]
    cnt = jnp.maximum(part_ref[0, :, H:H + 1] + part_ref[1, :, H:H + 1], 1.0)
    agg = (part_ref[0, :, :H] + part_ref[1, :, :H]) * (1.0 / cnt)
    m = jnp.maximum(agg + _dot(h, wr_ref[...]) + bconv_ref[...], 0.0)
    r = jax.nn.sigmoid(_dot(m, wir_ref[...]) + bir_ref[...]
                       + _dot(h, whr_ref[...]) + bhr_ref[...])
    zz = jax.nn.sigmoid(_dot(m, wiz_ref[...]) + biz_ref[...]
                        + _dot(h, whz_ref[...]) + bhz_ref[...])
    hn = _dot(h, whn_ref[...]) + bhn_ref[...]
    cand = jnp.tanh(_dot(m, win_ref[...]) + bin_ref[...] + r * hn)
    hout_ref[...] = _rep4((1.0 - zz) * cand + zz * h)


def _tc_update(h, parts, wr, bconv2, gru_ws, gru_bs):
    wspec = pl.BlockSpec((H, H), lambda i: (0, 0))
    bspec = pl.BlockSpec((1, H), lambda i: (0, 0))
    return pl.pallas_call(
        _upd_body,
        grid=(N // NB,),
        in_specs=[
            pl.BlockSpec((NB, W128), lambda i: (i, 0)),
            pl.BlockSpec((NC, NB, W128), lambda i: (0, i, 0)),
            wspec, bspec,
            wspec, wspec, wspec, wspec, wspec, wspec,
            bspec, bspec, bspec, bspec, bspec, bspec,
        ],
        out_specs=pl.BlockSpec((NB, W128), lambda i: (i, 0)),
        out_shape=jax.ShapeDtypeStruct((N, W128), jnp.float32),
    )(h, parts, wr, bconv2, *gru_ws, *gru_bs)


def _mlp_body(hs_ref, hd_ref, ea_ref, wa_ref, wb_ref, wc_ref, b1_ref,
              w2_ref, b2_ref, w3_ref, b3_ref, out_ref):
    d1 = jnp.maximum(_dot(hs_ref[:, :H], wa_ref[...])
                     + _dot(hd_ref[:, :H], wb_ref[...])
                     + _dot(ea_ref[...], wc_ref[...]) + b1_ref[...], 0.0)
    d2 = jnp.maximum(_dot(d1, w2_ref[...]) + b2_ref[...], 0.0)
    out_ref[...] = _dot(d2, w3_ref[...]) + b3_ref[...]


def _tc_mlp(hs, hd, ea, wa, wb, wc, b1_2, w2, b2_2, w3, b3_2, num_targets):
    return pl.pallas_call(
        _mlp_body,
        grid=(E // EB2,),
        in_specs=[
            pl.BlockSpec((EB2, W128), lambda i: (i, 0)),
            pl.BlockSpec((EB2, W128), lambda i: (i, 0)),
            pl.BlockSpec((EB2, EDGE_DIM), lambda i: (i, 0)),
            pl.BlockSpec((H, H), lambda i: (0, 0)),
            pl.BlockSpec((H, H), lambda i: (0, 0)),
            pl.BlockSpec((EDGE_DIM, H), lambda i: (0, 0)),
            pl.BlockSpec((1, H), lambda i: (0, 0)),
            pl.BlockSpec((H, H // 2), lambda i: (0, 0)),
            pl.BlockSpec((1, H // 2), lambda i: (0, 0)),
            pl.BlockSpec((H // 2, num_targets), lambda i: (0, 0)),
            pl.BlockSpec((1, num_targets), lambda i: (0, 0)),
        ],
        out_specs=pl.BlockSpec((EB2, num_targets), lambda i: (i, 0)),
        out_shape=jax.ShapeDtypeStruct((E, num_targets), jnp.float32),
    )(hs, hd, ea, wa, wb, wc, b1_2, w2, b2_2, w3, b3_2)


def kernel(x, edge_index, edge_attr, u, W_in, b_in, We1, be1, We2, be2,
           W_root, b_conv, W_ih, W_hh, b_ih, b_hh,
           Wd1, bd1, Wd2, bd2, Wd3, bd3):
    num_targets = Wd3.shape[1]
    src = edge_index[0]
    dst = edge_index[1]

    w2p = We2.reshape(HH, H)
    be2m = be2.reshape(H, H)
    we1rep = jnp.repeat(We1, H, axis=1)
    be1rep = jnp.repeat(be1, H).reshape(1, HH)
    wx, wu = W_in[:NODE_DIM], W_in[NODE_DIM:]
    b_in2 = b_in.reshape(1, H)
    bconv2 = b_conv.reshape(1, H)
    wihT = W_ih.T
    whhT = W_hh.T
    gru_ws = (wihT[:, :H], wihT[:, H:2 * H], wihT[:, 2 * H:],
              whhT[:, :H], whhT[:, H:2 * H], whhT[:, 2 * H:])
    gru_bs = (b_ih[:H].reshape(1, H), b_ih[H:2 * H].reshape(1, H),
              b_ih[2 * H:].reshape(1, H), b_hh[:H].reshape(1, H),
              b_hh[H:2 * H].reshape(1, H), b_hh[2 * H:].reshape(1, H))
    wr = W_root
    w2p_bf = w2p.astype(jnp.bfloat16)
    zeros_nw = jnp.zeros((N, W128), jnp.float32)
    wa, wb, wc = Wd1[:H], Wd1[H:2 * H], Wd1[2 * H:]

    h = _tc_prep(x, u, wx, wu, b_in2)

    for _ in range(3):
        hs = _sc_gather(h, src)
        msg = _tc_msg(edge_attr, hs, we1rep, be1rep, w2p_bf, be2m)
        parts = _sc_scatter(msg, dst, zeros_nw)
        h = _tc_update(h, parts, wr, bconv2, gru_ws, gru_bs)

    hs, hd = _sc_gather2(h, src, dst)
    return _tc_mlp(hs, hd, edge_attr, wa, wb, wc, bd1.reshape(1, H),
                   Wd2, bd2.reshape(1, H // 2), Wd3,
                   bd3.reshape(1, num_targets), num_targets)


# speedup vs baseline: 3.3534x; 1.0487x over previous
import jax
import jax.numpy as jnp
from jax import lax
from jax.experimental import pallas as pl
from jax.experimental.pallas import tpu as pltpu
from jax.experimental.pallas import tpu_sc as plsc

N = 10000
E = 160000
H = 32
HH = H * H
W128 = 128
NODE_DIM = 12
GLOBAL_DIM = 11
EDGE_DIM = 5

NC, NS = 2, 16
NW = NC * NS
CH = 128
NCHUNK = E // CH
CPW = NCHUNK // NW
CEXTRA = NCHUNK - CPW * NW
STRIPE = 632

EB = 4000
EB2 = 4000
NB = 1000

_MESH = plsc.VectorSubcoreMesh(
    core_axis_name="c", subcore_axis_name="s", num_cores=NC, num_subcores=NS)


def _dot(a, b):
    return jnp.dot(a, b, preferred_element_type=jnp.float32)


def _rep4(v):
    return jnp.concatenate([v, v, v, v], axis=1)


def _pad128_count(v, rows):
    return jnp.concatenate(
        [v, jnp.ones((rows, 1), jnp.float32),
         jnp.zeros((rows, W128 - H - 1), jnp.float32)], axis=1)


RING = 4
NPIPE = (CPW // RING) * RING


def _gather_pipe(table_hbm, idx_hbm, out_hbm, idxs, rowss, semi, semg, semo, w):
    def off(j):
        return (j * NW + w) * CH

    for b in range(2):
        pltpu.async_copy(idx_hbm.at[pl.ds(off(b), CH)], idxs[b], semi[b])

    @pl.loop(0, NPIPE // RING)
    def _grp(g):
        for t in range(RING):
            b = t
            b2 = (t + 2) % RING
            j = g * RING + t
            pltpu.make_async_copy(
                idx_hbm.at[pl.ds(0, CH)], idxs[b], semi[b]).wait()

            @pl.when(j >= RING)
            def _():
                pltpu.make_async_copy(
                    rowss[b], out_hbm.at[pl.ds(0, CH)], semo[b]).wait()

            pltpu.async_copy(table_hbm.at[idxs[b]], rowss[b], semg[b])

            @pl.when(j >= 2)
            def _():
                pltpu.make_async_copy(
                    table_hbm.at[pl.ds(0, CH)], rowss[b2], semg[b2]).wait()

            @pl.when(j + 2 < NPIPE)
            def _():
                pltpu.async_copy(
                    idx_hbm.at[pl.ds(off(j + 2), CH)], idxs[b2], semi[b2])

            @pl.when(j >= 2)
            def _():
                pltpu.async_copy(
                    rowss[b2], out_hbm.at[pl.ds(off(j - 2), CH)], semo[b2])

    for j in (NPIPE - 2, NPIPE - 1):
        b = j % RING
        pltpu.make_async_copy(
            table_hbm.at[pl.ds(0, CH)], rowss[b], semg[b]).wait()
        pltpu.async_copy(rowss[b], out_hbm.at[pl.ds(off(j), CH)], semo[b])
    for b in range(RING):
        pltpu.make_async_copy(
            rowss[b], out_hbm.at[pl.ds(0, CH)], semo[b]).wait()
    for j in range(NPIPE, CPW):
        pltpu.sync_copy(idx_hbm.at[pl.ds(off(j), CH)], idxs[0])
        pltpu.async_copy(table_hbm.at[idxs[0]], rowss[0], semg[0]).wait()
        pltpu.sync_copy(rowss[0], out_hbm.at[pl.ds(off(j), CH)])

    @pl.when(w < CEXTRA)
    def _extra():
        o = (CPW * NW + w) * CH
        pltpu.sync_copy(idx_hbm.at[pl.ds(o, CH)], idxs[0])
        pltpu.async_copy(table_hbm.at[idxs[0]], rowss[0], semg[0]).wait()
        pltpu.sync_copy(rowss[0], out_hbm.at[pl.ds(o, CH)])


_GATHER_SCRATCH = (
    [pltpu.VMEM((CH,), jnp.int32) for _ in range(RING)]
    + [pltpu.VMEM((CH, W128), jnp.float32) for _ in range(RING)]
    + [pltpu.SemaphoreType.DMA for _ in range(3 * RING)]
)


def _gather_body(table_hbm, idx_hbm, out_hbm, *scr):
    idxs, rowss = scr[:RING], scr[RING:2 * RING]
    semi, semg, semo = (scr[2 * RING:3 * RING], scr[3 * RING:4 * RING],
                        scr[4 * RING:5 * RING])
    w = lax.axis_index("c") * NS + lax.axis_index("s")
    _gather_pipe(table_hbm, idx_hbm, out_hbm, idxs, rowss, semi, semg, semo, w)


def _sc_gather(table, idx):
    return pl.kernel(
        _gather_body,
        out_type=jax.ShapeDtypeStruct((E, W128), jnp.float32),
        mesh=_MESH,
        scratch_types=list(_GATHER_SCRATCH),
    )(table, idx)


SRING = 2
SPIPE = (CPW // SRING) * SRING


def _scatter_body(msg_hbm, idx_hbm, zeros_hbm, out_hbm, *scr):
    idxs, rowss = scr[:SRING], scr[SRING:2 * SRING]
    semi, semm, sems = (scr[2 * SRING:3 * SRING], scr[3 * SRING:4 * SRING],
                        scr[4 * SRING:5 * SRING])
    acc_sh = scr[5 * SRING]
    c = lax.axis_index("c")
    s = lax.axis_index("s")
    w = c * NS + s
    stripe_off = jnp.minimum(s * STRIPE, N - STRIPE)
    pltpu.sync_copy(zeros_hbm.at[pl.ds(stripe_off, STRIPE)],
                    acc_sh.at[pl.ds(stripe_off, STRIPE)])

    def off(j):
        return (j * NW + w) * CH

    plsc.subcore_barrier()
    pltpu.async_copy(idx_hbm.at[pl.ds(off(0), CH)], idxs[0], semi[0])
    pltpu.async_copy(msg_hbm.at[pl.ds(off(0), CH)], rowss[0], semm[0])

    @pl.loop(0, SPIPE // SRING)
    def _grp(g):
        for t in range(SRING):
            b = t
            b2 = t ^ 1
            j = g * SRING + t
            pltpu.make_async_copy(
                idx_hbm.at[pl.ds(0, CH)], idxs[b], semi[b]).wait()
            pltpu.make_async_copy(
                msg_hbm.at[pl.ds(0, CH)], rowss[b], semm[b]).wait()
            pltpu.async_copy(rowss[b], acc_sh.at[idxs[b]], sems[b], add=True)

            @pl.when(j >= 1)
            def _():
                pltpu.make_async_copy(
                    rowss[b2], acc_sh.at[pl.ds(0, CH)], sems[b2]).wait()

            @pl.when(j + 1 < SPIPE)
            def _():
                pltpu.async_copy(
                    idx_hbm.at[pl.ds(off(j + 1), CH)], idxs[b2], semi[b2])
                pltpu.async_copy(
                    msg_hbm.at[pl.ds(off(j + 1), CH)], rowss[b2], semm[b2])

    b_last = (SPIPE - 1) % SRING
    pltpu.make_async_copy(
        rowss[b_last], acc_sh.at[pl.ds(0, CH)], sems[b_last]).wait()
    for j in range(SPIPE, CPW):
        pltpu.sync_copy(idx_hbm.at[pl.ds(off(j), CH)], idxs[0])
        pltpu.sync_copy(msg_hbm.at[pl.ds(off(j), CH)], rowss[0])
        pltpu.sync_copy(rowss[0], acc_sh.at[idxs[0]], add=True)

    @pl.when(w < CEXTRA)
    def _extra():
        o = (CPW * NW + w) * CH
        pltpu.sync_copy(idx_hbm.at[pl.ds(o, CH)], idxs[0])
        pltpu.sync_copy(msg_hbm.at[pl.ds(o, CH)], rowss[0])
        pltpu.sync_copy(rowss[0], acc_sh.at[idxs[0]], add=True)

    plsc.subcore_barrier()
    pltpu.sync_copy(acc_sh.at[pl.ds(stripe_off, STRIPE)],
                    out_hbm.at[c].at[pl.ds(stripe_off, STRIPE)])


_SCATTER_SCRATCH = (
    [pltpu.VMEM((CH,), jnp.int32) for _ in range(SRING)]
    + [pltpu.VMEM((CH, W128), jnp.float32) for _ in range(SRING)]
    + [pltpu.SemaphoreType.DMA for _ in range(3 * SRING)]
    + [pltpu.VMEM_SHARED((N, W128), jnp.float32)]
)


def _sc_scatter(msg, idx, zeros_nw):
    return pl.kernel(
        _scatter_body,
        out_type=jax.ShapeDtypeStruct((NC, N, W128), jnp.float32),
        mesh=_MESH,
        scratch_types=list(_SCATTER_SCRATCH),
    )(msg, idx, zeros_nw)


def _gather2_body(table_hbm, idx1_hbm, idx2_hbm, out1_hbm, out2_hbm, *scr):
    idxs, rowss = scr[:RING], scr[RING:2 * RING]
    semi, semg, semo = (scr[2 * RING:3 * RING], scr[3 * RING:4 * RING],
                        scr[4 * RING:5 * RING])
    w = lax.axis_index("c") * NS + lax.axis_index("s")
    _gather_pipe(table_hbm, idx1_hbm, out1_hbm, idxs, rowss,
                 semi, semg, semo, w)
    _gather_pipe(table_hbm, idx2_hbm, out2_hbm, idxs, rowss,
                 semi, semg, semo, w)


def _sc_gather2(table, idx1, idx2):
    return pl.kernel(
        _gather2_body,
        out_type=(jax.ShapeDtypeStruct((E, W128), jnp.float32),
                  jax.ShapeDtypeStruct((E, W128), jnp.float32)),
        mesh=_MESH,
        scratch_types=list(_GATHER_SCRATCH),
    )(table, idx1, idx2)


def _prep_body(x_ref, u_ref, wx_ref, wu_ref, b_ref, h0_ref):
    ub = _dot(u_ref[...], wu_ref[...]) + b_ref[...]
    h0 = jnp.tanh(_dot(x_ref[...], wx_ref[...]) + ub)
    h0_ref[...] = _rep4(h0)


def _tc_prep(x, u, wx, wu, b_in2):
    return pl.pallas_call(
        _prep_body,
        grid=(N // NB,),
        in_specs=[
            pl.BlockSpec((NB, NODE_DIM), lambda i: (i, 0)),
            pl.BlockSpec((1, GLOBAL_DIM), lambda i: (0, 0)),
            pl.BlockSpec((NODE_DIM, H), lambda i: (0, 0)),
            pl.BlockSpec((GLOBAL_DIM, H), lambda i: (0, 0)),
            pl.BlockSpec((1, H), lambda i: (0, 0)),
        ],
        out_specs=pl.BlockSpec((NB, W128), lambda i: (i, 0)),
        out_shape=jax.ShapeDtypeStruct((N, W128), jnp.float32),
    )(x, u, wx, wu, b_in2)


def _msg_body(ea_ref, hs_ref, we1rep_ref, be1rep_ref, w2p_ref, be2m_ref,
              msg_ref):
    hs = hs_ref[:, :H]
    hs4 = hs_ref[...]
    a = jnp.maximum(_dot(ea_ref[...], we1rep_ref[...]) + be1rep_ref[...], 0.0)
    p = jnp.concatenate(
        [(a[:, i * W128:(i + 1) * W128] * hs4).astype(jnp.bfloat16)
         for i in range(HH // W128)], axis=1)
    msg = (jnp.dot(p, w2p_ref[...], preferred_element_type=jnp.float32)
           + _dot(hs, be2m_ref[...]))
    msg_ref[...] = _pad128_count(msg, EB)


def _tc_msg(ea, hs, we1rep, be1rep, w2p_bf, be2m):
    return pl.pallas_call(
        _msg_body,
        grid=(E // EB,),
        in_specs=[
            pl.BlockSpec((EB, EDGE_DIM), lambda i: (i, 0)),
            pl.BlockSpec((EB, W128), lambda i: (i, 0)),
            pl.BlockSpec((EDGE_DIM, HH), lambda i: (0, 0)),
            pl.BlockSpec((1, HH), lambda i: (0, 0)),
            pl.BlockSpec((HH, H), lambda i: (0, 0)),
            pl.BlockSpec((H, H), lambda i: (0, 0)),
        ],
        out_specs=pl.BlockSpec((EB, W128), lambda i: (i, 0)),
        out_shape=jax.ShapeDtypeStruct((E, W128), jnp.float32),
    )(ea, hs, we1rep, be1rep, w2p_bf, be2m)


def _upd_body(h_ref, part_ref, wr_ref, bconv_ref,
              wir_ref, wiz_ref, win_ref, whr_ref, whz_ref, whn_ref,
              bir_ref, biz_ref, bin_ref, bhr_ref, bhz_ref, bhn_ref,
              hout_ref):
    h = h_ref[:, :H]
    cnt = jnp.maximum(part_ref[0, :, H:H + 1] + part_ref[1, :, H:H + 1], 1.0)
    agg = (part_ref[0, :, :H] + part_ref[1, :, :H]) * (1.0 / cnt)
    m = jnp.maximum(agg + _dot(h, wr_ref[...]) + bconv_ref[...], 0.0)
    r = jax.nn.sigmoid(_dot(m, wir_ref[...]) + bir_ref[...]
                       + _dot(h, whr_ref[...]) + bhr_ref[...])
    zz = jax.nn.sigmoid(_dot(m, wiz_ref[...]) + biz_ref[...]
                        + _dot(h, whz_ref[...]) + bhz_ref[...])
    hn = _dot(h, whn_ref[...]) + bhn_ref[...]
    cand = jnp.tanh(_dot(m, win_ref[...]) + bin_ref[...] + r * hn)
    hout_ref[...] = _rep4((1.0 - zz) * cand + zz * h)


def _tc_update(h, parts, wr, bconv2, gru_ws, gru_bs):
    wspec = pl.BlockSpec((H, H), lambda i: (0, 0))
    bspec = pl.BlockSpec((1, H), lambda i: (0, 0))
    return pl.pallas_call(
        _upd_body,
        grid=(N // NB,),
        in_specs=[
            pl.BlockSpec((NB, W128), lambda i: (i, 0)),
            pl.BlockSpec((NC, NB, W128), lambda i: (0, i, 0)),
            wspec, bspec,
            wspec, wspec, wspec, wspec, wspec, wspec,
            bspec, bspec, bspec, bspec, bspec, bspec,
        ],
        out_specs=pl.BlockSpec((NB, W128), lambda i: (i, 0)),
        out_shape=jax.ShapeDtypeStruct((N, W128), jnp.float32),
    )(h, parts, wr, bconv2, *gru_ws, *gru_bs)


def _mlp_body(hs_ref, hd_ref, ea_ref, wa_ref, wb_ref, wc_ref, b1_ref,
              w2_ref, b2_ref, w3_ref, b3_ref, out_ref):
    d1 = jnp.maximum(_dot(hs_ref[:, :H], wa_ref[...])
                     + _dot(hd_ref[:, :H], wb_ref[...])
                     + _dot(ea_ref[...], wc_ref[...]) + b1_ref[...], 0.0)
    d2 = jnp.maximum(_dot(d1, w2_ref[...]) + b2_ref[...], 0.0)
    out_ref[...] = _dot(d2, w3_ref[...]) + b3_ref[...]


def _tc_mlp(hs, hd, ea, wa, wb, wc, b1_2, w2, b2_2, w3, b3_2, num_targets):
    return pl.pallas_call(
        _mlp_body,
        grid=(E // EB2,),
        in_specs=[
            pl.BlockSpec((EB2, W128), lambda i: (i, 0)),
            pl.BlockSpec((EB2, W128), lambda i: (i, 0)),
            pl.BlockSpec((EB2, EDGE_DIM), lambda i: (i, 0)),
            pl.BlockSpec((H, H), lambda i: (0, 0)),
            pl.BlockSpec((H, H), lambda i: (0, 0)),
            pl.BlockSpec((EDGE_DIM, H), lambda i: (0, 0)),
            pl.BlockSpec((1, H), lambda i: (0, 0)),
            pl.BlockSpec((H, H // 2), lambda i: (0, 0)),
            pl.BlockSpec((1, H // 2), lambda i: (0, 0)),
            pl.BlockSpec((H // 2, num_targets), lambda i: (0, 0)),
            pl.BlockSpec((1, num_targets), lambda i: (0, 0)),
        ],
        out_specs=pl.BlockSpec((EB2, num_targets), lambda i: (i, 0)),
        out_shape=jax.ShapeDtypeStruct((E, num_targets), jnp.float32),
    )(hs, hd, ea, wa, wb, wc, b1_2, w2, b2_2, w3, b3_2)


def kernel(x, edge_index, edge_attr, u, W_in, b_in, We1, be1, We2, be2,
           W_root, b_conv, W_ih, W_hh, b_ih, b_hh,
           Wd1, bd1, Wd2, bd2, Wd3, bd3):
    num_targets = Wd3.shape[1]
    src = edge_index[0]
    dst = edge_index[1]

    w2p = We2.reshape(HH, H)
    be2m = be2.reshape(H, H)
    we1rep = jnp.repeat(We1, H, axis=1)
    be1rep = jnp.repeat(be1, H).reshape(1, HH)
    wx, wu = W_in[:NODE_DIM], W_in[NODE_DIM:]
    b_in2 = b_in.reshape(1, H)
    bconv2 = b_conv.reshape(1, H)
    wihT = W_ih.T
    whhT = W_hh.T
    gru_ws = (wihT[:, :H], wihT[:, H:2 * H], wihT[:, 2 * H:],
              whhT[:, :H], whhT[:, H:2 * H], whhT[:, 2 * H:])
    gru_bs = (b_ih[:H].reshape(1, H), b_ih[H:2 * H].reshape(1, H),
              b_ih[2 * H:].reshape(1, H), b_hh[:H].reshape(1, H),
              b_hh[H:2 * H].reshape(1, H), b_hh[2 * H:].reshape(1, H))
    wr = W_root
    w2p_bf = w2p.astype(jnp.bfloat16)
    zeros_nw = jnp.zeros((N, W128), jnp.float32)
    wa, wb, wc = Wd1[:H], Wd1[H:2 * H], Wd1[2 * H:]

    h = _tc_prep(x, u, wx, wu, b_in2)

    for _ in range(3):
        hs = _sc_gather(h, src)
        msg = _tc_msg(edge_attr, hs, we1rep, be1rep, w2p_bf, be2m)
        parts = _sc_scatter(msg, dst, zeros_nw)
        h = _tc_update(h, parts, wr, bconv2, gru_ws, gru_bs)

    hs, hd = _sc_gather2(h, src, dst)
    return _tc_mlp(hs, hd, edge_attr, wa, wb, wc, bd1.reshape(1, H),
                   Wd2, bd2.reshape(1, H // 2), Wd3,
                   bd3.reshape(1, num_targets), num_targets)
